# baseline (speedup 1.0000x reference)
import functools

import jax
import jax.numpy as jnp
from jax import lax
from jax.experimental import pallas as pl
from jax.experimental.pallas import tpu as pltpu
from jax.experimental.pallas import tpu_sc as plsc

N0, N1, N2, D = 100000, 25000, 6250, 128
L = 16
NW = 32
BLK = 1000

N2P, N1P = 7000, 26000
T1P = 25088
T0P = 100352
SENT = 1 << 26

_MESH = dict(core_axis_name="c", subcore_axis_name="s")


def _wid():
    return lax.axis_index("s") * 2 + lax.axis_index("c")


def _lane():
    return lax.broadcasted_iota(jnp.int32, (L,), 0)


def _splat(x):
    return jnp.broadcast_to(x, (L,)).astype(jnp.int32)


def _invert_body(idx_hbm, tl128_hbm, jl128_hbm, tl16_hbm, jl16_hbm, cnt_hbm,
                 idx_buf, jw_buf, tl128, jl128, tl16, jl16, cntbuf,
                 *, n, tpb, ch128, ch16, zbase):
    wid = _wid()
    lo = wid * tpb
    lane = _lane()
    zvec = zbase + wid * L + lane

    pltpu.sync_copy(idx_hbm, idx_buf.at[pl.ds(0, n)])
    nfloor = n - n % L
    tail = idx_buf[pl.ds(nfloor, L)]
    idx_buf[pl.ds(nfloor, L)] = jnp.where(lane < n - nfloor, tail, SENT)
    idx_buf[pl.ds(nfloor + L, L)] = jnp.full((L,), SENT, jnp.int32)

    minus1 = jnp.full((L,), -1, jnp.int32)

    def init_jw(i, _):
        jw_buf[pl.ds(i * L, L)] = minus1
        return 0

    lax.fori_loop(0, tpb // L, init_jw, 0)

    def init_lists(i, _):
        for c in range(8):
            tl128[i, pl.ds(c * L, L)] = jnp.zeros((L,), jnp.int32)
            jl128[i, pl.ds(c * L, L)] = zvec
        return 0

    lax.fori_loop(0, ch128, init_lists, 0)

    def init_lists16(i, _):
        tl16[i, :] = jnp.zeros((L,), jnp.int32)
        jl16[i, :] = zvec
        return 0

    lax.fori_loop(0, ch16, init_lists16, 0)

    def scan_pair(c, _):
        chunks = (2 * c, 2 * c + 1)
        stores = []
        for cc in chunks:
            v = idx_buf[pl.ds(cc * L, L)]
            m = (v >= lo) & (v < lo + tpb)
            iv = jnp.where(m, v - lo, 0)
            jval = cc * L + lane
            plsc.store_scatter(jw_buf, [iv], jval, mask=m)
            stores.append((iv, m, jval))
        for iv, m, jval in stores:
            def fix(_, iv=iv, m=m, jval=jval):
                got = plsc.load_gather(jw_buf, [iv], mask=m)
                lost = m & (got < jval)
                plsc.store_scatter(jw_buf, [iv], jval, mask=lost)
                return jnp.any(lost)
            lax.while_loop(lambda a: a, fix, jnp.bool_(True))
        return 0

    lax.fori_loop(0, (n + 2 * L - 1) // (2 * L), scan_pair, 0)

    def comp_body(i, off):
        v = jw_buf[pl.ds(i * L, L)]
        m = v >= 0
        mi = m.astype(jnp.int32)
        pos = off + plsc.cumsum(mi) - 1
        t = lo + i * L + lane
        plsc.store_scatter(tl128, [pos >> 7, pos & 127], t, mask=m)
        plsc.store_scatter(jl128, [pos >> 7, pos & 127], v, mask=m)
        plsc.store_scatter(tl16, [pos >> 4, pos & 15], t, mask=m)
        plsc.store_scatter(jl16, [pos >> 4, pos & 15], v, mask=m)
        return off + jnp.sum(mi)

    cnt = lax.fori_loop(0, tpb // L, comp_body, jnp.int32(0))

    cntbuf[...] = _splat(cnt)
    pltpu.sync_copy(tl128, tl128_hbm.at[wid])
    pltpu.sync_copy(jl128, jl128_hbm.at[wid])
    pltpu.sync_copy(tl16, tl16_hbm.at[wid])
    pltpu.sync_copy(jl16, jl16_hbm.at[wid])
    pltpu.sync_copy(cntbuf, cnt_hbm.at[wid])


def _invert(idx, n, tp, zbase):
    tpb = tp // NW
    ch128 = (tpb + 127) // 128
    ch16 = tpb // L
    body = functools.partial(_invert_body, n=n, tpb=tpb, ch128=ch128,
                             ch16=ch16, zbase=zbase)
    i32 = jnp.int32
    return pl.kernel(
        body,
        out_type=(
            jax.ShapeDtypeStruct((NW, ch128, 128), i32),
            jax.ShapeDtypeStruct((NW, ch128, 128), i32),
            jax.ShapeDtypeStruct((NW, ch16, L), i32),
            jax.ShapeDtypeStruct((NW, ch16, L), i32),
            jax.ShapeDtypeStruct((NW, L), i32),
        ),
        mesh=plsc.VectorSubcoreMesh(**_MESH),
        scratch_types=[
            pltpu.VMEM((25024,), i32),
            pltpu.VMEM((tpb,), i32),
            pltpu.VMEM((ch128, 128), i32),
            pltpu.VMEM((ch128, 128), i32),
            pltpu.VMEM((ch16, L), i32),
            pltpu.VMEM((ch16, L), i32),
            pltpu.VMEM((L,), i32),
        ],
        compiler_params=pltpu.CompilerParams(needs_layout_passes=False),
    )(idx)


def _rmw_body(tl128_hbm, jl128_hbm, tl16_hbm, jl16_hbm, cnt_hbm, g_hbm, h_ref,
              tl128, jl128, tl16, jl16, cntbuf, bufG, bufH, semG, semH, semS,
              *, ch128, ch16):
    wid = _wid()
    lane = _lane()
    pltpu.sync_copy(tl128_hbm.at[wid], tl128)
    pltpu.sync_copy(jl128_hbm.at[wid], jl128)
    pltpu.sync_copy(tl16_hbm.at[wid], tl16)
    pltpu.sync_copy(jl16_hbm.at[wid], jl16)
    pltpu.sync_copy(cnt_hbm.at[wid], cntbuf)
    cnt = jnp.max(cntbuf[...])
    n128 = cnt >> 7
    n16 = (cnt & 127) >> 4
    rem = cnt & 15

    def add_rows(nrows):
        def row_body(r, _):
            for c in range(8):
                s = pl.ds(c * L, L)
                bufH[r, s] = bufH[r, s] + bufG[r, s]
            return 0
        lax.fori_loop(0, nrows, row_body, 0)

    def big_chunk(k, _):
        gh = pltpu.async_copy(g_hbm.at[jl128.at[k]], bufG, semG)
        hh = pltpu.async_copy(h_ref.at[tl128.at[k]], bufH, semH)
        gh.wait()
        hh.wait()
        add_rows(128)
        pltpu.async_copy(bufH, h_ref.at[tl128.at[k]], semS).wait()
        return 0

    lax.fori_loop(0, n128, big_chunk, 0)

    def mid_chunk(i, _):
        k = n128 * 8 + i
        gh = pltpu.async_copy(g_hbm.at[jl16.at[k]], bufG.at[pl.ds(0, L)],
                              semG)
        hh = pltpu.async_copy(h_ref.at[tl16.at[k]], bufH.at[pl.ds(0, L)],
                              semH)
        gh.wait()
        hh.wait()
        add_rows(L)
        pltpu.async_copy(bufH.at[pl.ds(0, L)], h_ref.at[tl16.at[k]],
                         semS).wait()
        return 0

    lax.fori_loop(0, n16, mid_chunk, 0)

    ktail = n128 * 8 + n16
    tvec = plsc.load_gather(tl16, [_splat(ktail), lane])
    jvec = plsc.load_gather(jl16, [_splat(ktail), lane])

    def scalar_at(vec, e):
        return jnp.max(jnp.where(lane == e, vec, -1))

    for e in range(15):
        @pl.when(e < rem)
        def _start():
            j_e = scalar_at(jvec, e)
            t_e = scalar_at(tvec, e)
            pltpu.make_async_copy(g_hbm.at[pl.ds(j_e, 1)],
                                  bufG.at[pl.ds(e, 1)], semG).start()
            pltpu.make_async_copy(h_ref.at[pl.ds(t_e, 1)],
                                  bufH.at[pl.ds(e, 1)], semH).start()

    for e in range(15):
        @pl.when(e < rem)
        def _apply():
            pltpu.make_async_copy(g_hbm.at[pl.ds(0, 1)],
                                  bufG.at[pl.ds(e, 1)], semG).wait()
            pltpu.make_async_copy(g_hbm.at[pl.ds(0, 1)],
                                  bufH.at[pl.ds(e, 1)], semH).wait()
            for c in range(8):
                s = pl.ds(c * L, L)
                bufH[e, s] = bufH[e, s] + bufG[e, s]
            t_e = scalar_at(tvec, e)
            pltpu.make_async_copy(bufH.at[pl.ds(e, 1)],
                                  h_ref.at[pl.ds(t_e, 1)], semS).start()

    for e in range(15):
        @pl.when(e < rem)
        def _drain():
            pltpu.make_async_copy(bufH.at[pl.ds(e, 1)],
                                  h_ref.at[pl.ds(0, 1)], semS).wait()


def _rmw(lists, g, h_ref, tp):
    tpb = tp // NW
    ch128 = (tpb + 127) // 128
    ch16 = tpb // L
    body = functools.partial(_rmw_body, ch128=ch128, ch16=ch16)
    i32 = jnp.int32
    pl.kernel(
        body,
        out_type=(),
        mesh=plsc.VectorSubcoreMesh(**_MESH),
        scratch_types=[
            pltpu.VMEM((ch128, 128), i32),
            pltpu.VMEM((ch128, 128), i32),
            pltpu.VMEM((ch16, L), i32),
            pltpu.VMEM((ch16, L), i32),
            pltpu.VMEM((L,), i32),
            pltpu.VMEM((128, D), jnp.float32),
            pltpu.VMEM((128, D), jnp.float32),
            pltpu.SemaphoreType.DMA,
            pltpu.SemaphoreType.DMA,
            pltpu.SemaphoreType.DMA,
        ],
        compiler_params=pltpu.CompilerParams(needs_layout_passes=False),
    )(*lists, g, h_ref)


def _prep_body(w0, b0, bup0, w0b, w1, b1, bup1, w1b,
               w0e, b0e, w1e, b1e):
    f32 = jnp.float32
    w0e[...] = w0[...] + jnp.dot(w0[...], w0b[...], preferred_element_type=f32)
    b0e[...] = b0[...] + jnp.dot(b0[...], w0b[...], preferred_element_type=f32) + bup0[...]
    w1e[...] = w1[...] + jnp.dot(w1[...], w1b[...], preferred_element_type=f32)
    b1e[...] = b1[...] + jnp.dot(b1[...], w1b[...], preferred_element_type=f32) + bup1[...]


def _prep(W0, b0r, bup0r, Wup0b, W1, b1r, bup1r, Wup1b):
    f32 = jnp.float32
    return pl.pallas_call(
        _prep_body,
        out_shape=(
            jax.ShapeDtypeStruct((D, D), f32),
            jax.ShapeDtypeStruct((1, D), f32),
            jax.ShapeDtypeStruct((D, D), f32),
            jax.ShapeDtypeStruct((1, D), f32),
        ),
    )(W0, b0r, bup0r, Wup0b, W1, b1r, bup1r, Wup1b)


def _tc2_body(hn2_ref, w2_ref, b2_ref, wt_ref, h2_ref, g2_ref):
    g = pl.program_id(0)
    rows = g * BLK + lax.broadcasted_iota(jnp.int32, (BLK, 1), 0)
    h = jnp.dot(hn2_ref[...], w2_ref[...],
                preferred_element_type=jnp.float32) + b2_ref[...]
    h2_ref[...] = h
    gv = jnp.dot(h, wt_ref[...], preferred_element_type=jnp.float32)
    g2_ref[...] = jnp.where(rows < N2, gv, 0.0)


def _tc2(hn2, W2, b2r, Wup1t):
    nb = N2P // BLK
    return pl.pallas_call(
        _tc2_body,
        grid=(nb,),
        in_specs=[
            pl.BlockSpec((BLK, D), lambda i: (i, 0)),
            pl.BlockSpec((D, D), lambda i: (0, 0)),
            pl.BlockSpec((1, D), lambda i: (0, 0)),
            pl.BlockSpec((D, D), lambda i: (0, 0)),
        ],
        out_specs=(
            pl.BlockSpec((BLK, D), lambda i: (i, 0)),
            pl.BlockSpec((BLK, D), lambda i: (i, 0)),
        ),
        out_shape=(
            jax.ShapeDtypeStruct((N2, D), jnp.float32),
            jax.ShapeDtypeStruct((N2P, D), jnp.float32),
        ),
    )(hn2, W2, b2r, Wup1t)


def _dense_body(hn_ref, w_ref, b_ref, out_ref):
    out_ref[...] = jnp.dot(hn_ref[...], w_ref[...],
                           preferred_element_type=jnp.float32) + b_ref[...]


def _dense(hn, We, be, nrows):
    return pl.pallas_call(
        _dense_body,
        grid=(nrows // BLK,),
        in_specs=[
            pl.BlockSpec((BLK, D), lambda i: (i, 0)),
            pl.BlockSpec((D, D), lambda i: (0, 0)),
            pl.BlockSpec((1, D), lambda i: (0, 0)),
        ],
        out_specs=pl.BlockSpec((BLK, D), lambda i: (i, 0)),
        out_shape=jax.ShapeDtypeStruct((nrows, D), jnp.float32),
    )(hn, We, be)


def _g1_body(h1_ref, wt_ref, g1_ref):
    g = pl.program_id(0)
    rows = g * BLK + lax.broadcasted_iota(jnp.int32, (BLK, 1), 0)
    gv = jnp.dot(h1_ref[...], wt_ref[...], preferred_element_type=jnp.float32)
    g1_ref[...] = jnp.where(rows < N1, gv, 0.0)


def _g1(h1, Wup0t):
    nb = N1P // BLK
    return pl.pallas_call(
        _g1_body,
        grid=(nb,),
        in_specs=[
            pl.BlockSpec((BLK, D), lambda i: (jnp.minimum(i, 24), 0)),
            pl.BlockSpec((D, D), lambda i: (0, 0)),
        ],
        out_specs=pl.BlockSpec((BLK, D), lambda i: (i, 0)),
        out_shape=jax.ShapeDtypeStruct((N1P, D), jnp.float32),
    )(h1, Wup0t)


def kernel(hn0, hn1, hn2, idx1, idx2, W0, b0, W1, b1, W2, b2, Wup0, bup0,
           Wup1, bup1):
    b0r, b1r, b2r = b0.reshape(1, D), b1.reshape(1, D), b2.reshape(1, D)
    bup0r, bup1r = bup0.reshape(1, D), bup1.reshape(1, D)
    Wup0t, Wup0b = Wup0[:D], Wup0[D:]
    Wup1t, Wup1b = Wup1[:D], Wup1[D:]

    W0e, b0e, W1e, b1e = _prep(W0, b0r, bup0r, Wup0b, W1, b1r, bup1r, Wup1b)

    lists2 = _invert(idx2, N2, T1P, zbase=N2)
    lists1 = _invert(idx1, N1, T0P, zbase=N1)

    h2, g2pad = _tc2(hn2, W2, b2r, Wup1t)

    h1d = _dense(hn1, W1e, b1e, N1)
    h1_ref = jax.new_ref(h1d)
    _rmw(lists2, g2pad, h1_ref, T1P)
    h1 = jax.freeze(h1_ref)

    g1pad = _g1(h1, Wup0t)

    h0d = _dense(hn0, W0e, b0e, N0)
    h0_ref = jax.new_ref(h0d)
    _rmw(lists1, g1pad, h0_ref, T0P)
    h0 = jax.freeze(h0_ref)

    return (h0, h1, h2)


# speedup vs baseline: 1.0022x; 1.0022x over previous
import functools

import jax
import jax.numpy as jnp
from jax import lax
from jax.experimental import pallas as pl
from jax.experimental.pallas import tpu as pltpu
from jax.experimental.pallas import tpu_sc as plsc

N0, N1, N2, D = 100000, 25000, 6250, 128
L = 16
NW = 32
BLK = 1000

N2P, N1P = 7000, 26000
T1P = 25088
T0P = 100352
SENT = 1 << 26

_MESH = dict(core_axis_name="c", subcore_axis_name="s")


def _wid():
    return lax.axis_index("s") * 2 + lax.axis_index("c")


def _lane():
    return lax.broadcasted_iota(jnp.int32, (L,), 0)


def _splat(x):
    return jnp.broadcast_to(x, (L,)).astype(jnp.int32)


def _invert_body(idx_hbm, tl128_hbm, jl128_hbm, tl16_hbm, jl16_hbm, cnt_hbm,
                 idx_buf, jw_buf, tl128, jl128, tl16, jl16, cntbuf,
                 *, n, tpb, ch128, ch16, zbase):
    wid = _wid()
    lo = wid * tpb
    lane = _lane()
    zvec = zbase + wid * L + lane

    pltpu.sync_copy(idx_hbm, idx_buf.at[pl.ds(0, n)])
    nfloor = n - n % L
    tail = idx_buf[pl.ds(nfloor, L)]
    idx_buf[pl.ds(nfloor, L)] = jnp.where(lane < n - nfloor, tail, SENT)
    idx_buf[pl.ds(nfloor + L, L)] = jnp.full((L,), SENT, jnp.int32)

    minus1 = jnp.full((L,), -1, jnp.int32)

    def init_jw(i, _):
        jw_buf[pl.ds(i * L, L)] = minus1
        return 0

    lax.fori_loop(0, tpb // L, init_jw, 0)

    def init_lists(i, _):
        for c in range(8):
            tl128[i, pl.ds(c * L, L)] = jnp.zeros((L,), jnp.int32)
            jl128[i, pl.ds(c * L, L)] = zvec
        return 0

    lax.fori_loop(0, ch128, init_lists, 0)

    def init_lists16(i, _):
        tl16[i, :] = jnp.zeros((L,), jnp.int32)
        jl16[i, :] = zvec
        return 0

    lax.fori_loop(0, ch16, init_lists16, 0)

    perm = jnp.minimum(lane + 1, L - 1)

    def scan_body(c, _):
        v = idx_buf[pl.ds(c * L, L)]
        key = v * L + lane
        skey = plsc.sort_key_val(key, key)[0]
        tgt = skey >> 4
        nxt = skey.at[perm].get(mode="promise_in_bounds")
        keep = (tgt != (nxt >> 4)) | (lane == L - 1)
        m = keep & (tgt >= lo) & (tgt < lo + tpb)
        jval = c * L + (skey & (L - 1))
        plsc.store_scatter(jw_buf, [tgt - lo], jval, mask=m)
        return 0

    lax.fori_loop(0, (n + L - 1) // L, scan_body, 0)

    def comp_body(i, off):
        v = jw_buf[pl.ds(i * L, L)]
        m = v >= 0
        mi = m.astype(jnp.int32)
        pos = off + plsc.cumsum(mi) - 1
        t = lo + i * L + lane
        plsc.store_scatter(tl128, [pos >> 7, pos & 127], t, mask=m)
        plsc.store_scatter(jl128, [pos >> 7, pos & 127], v, mask=m)
        plsc.store_scatter(tl16, [pos >> 4, pos & 15], t, mask=m)
        plsc.store_scatter(jl16, [pos >> 4, pos & 15], v, mask=m)
        return off + jnp.sum(mi)

    cnt = lax.fori_loop(0, tpb // L, comp_body, jnp.int32(0))

    cntbuf[...] = _splat(cnt)
    pltpu.sync_copy(tl128, tl128_hbm.at[wid])
    pltpu.sync_copy(jl128, jl128_hbm.at[wid])
    pltpu.sync_copy(tl16, tl16_hbm.at[wid])
    pltpu.sync_copy(jl16, jl16_hbm.at[wid])
    pltpu.sync_copy(cntbuf, cnt_hbm.at[wid])


def _invert(idx, n, tp, zbase):
    tpb = tp // NW
    ch128 = (tpb + 127) // 128
    ch16 = tpb // L
    body = functools.partial(_invert_body, n=n, tpb=tpb, ch128=ch128,
                             ch16=ch16, zbase=zbase)
    i32 = jnp.int32
    return pl.kernel(
        body,
        out_type=(
            jax.ShapeDtypeStruct((NW, ch128, 128), i32),
            jax.ShapeDtypeStruct((NW, ch128, 128), i32),
            jax.ShapeDtypeStruct((NW, ch16, L), i32),
            jax.ShapeDtypeStruct((NW, ch16, L), i32),
            jax.ShapeDtypeStruct((NW, L), i32),
        ),
        mesh=plsc.VectorSubcoreMesh(**_MESH),
        scratch_types=[
            pltpu.VMEM((25024,), i32),
            pltpu.VMEM((tpb,), i32),
            pltpu.VMEM((ch128, 128), i32),
            pltpu.VMEM((ch128, 128), i32),
            pltpu.VMEM((ch16, L), i32),
            pltpu.VMEM((ch16, L), i32),
            pltpu.VMEM((L,), i32),
        ],
        compiler_params=pltpu.CompilerParams(needs_layout_passes=False, use_tc_tiling_on_sc=True),
    )(idx)


def _rmw_body(tl128_hbm, jl128_hbm, tl16_hbm, jl16_hbm, cnt_hbm, g_hbm, h_ref,
              tl128, jl128, tl16, jl16, cntbuf, bufG, bufH, semG, semH, semS,
              *, ch128, ch16):
    wid = _wid()
    lane = _lane()
    pltpu.sync_copy(tl128_hbm.at[wid], tl128)
    pltpu.sync_copy(jl128_hbm.at[wid], jl128)
    pltpu.sync_copy(tl16_hbm.at[wid], tl16)
    pltpu.sync_copy(jl16_hbm.at[wid], jl16)
    pltpu.sync_copy(cnt_hbm.at[wid], cntbuf)
    cnt = jnp.max(cntbuf[...])
    n128 = cnt >> 7
    n16 = (cnt & 127) >> 4
    rem = cnt & 15

    def add_rows(nrows):
        def row_body(r, _):
            for c in range(8):
                s = pl.ds(c * L, L)
                bufH[r, s] = bufH[r, s] + bufG[r, s]
            return 0
        lax.fori_loop(0, nrows, row_body, 0)

    def big_chunk(k, _):
        gh = pltpu.async_copy(g_hbm.at[jl128.at[k]], bufG, semG)
        hh = pltpu.async_copy(h_ref.at[tl128.at[k]], bufH, semH)
        gh.wait()
        hh.wait()
        add_rows(128)
        pltpu.async_copy(bufH, h_ref.at[tl128.at[k]], semS).wait()
        return 0

    lax.fori_loop(0, n128, big_chunk, 0)

    def mid_chunk(i, _):
        k = n128 * 8 + i
        gh = pltpu.async_copy(g_hbm.at[jl16.at[k]], bufG.at[pl.ds(0, L)],
                              semG)
        hh = pltpu.async_copy(h_ref.at[tl16.at[k]], bufH.at[pl.ds(0, L)],
                              semH)
        gh.wait()
        hh.wait()
        add_rows(L)
        pltpu.async_copy(bufH.at[pl.ds(0, L)], h_ref.at[tl16.at[k]],
                         semS).wait()
        return 0

    lax.fori_loop(0, n16, mid_chunk, 0)

    ktail = n128 * 8 + n16
    tvec = plsc.load_gather(tl16, [_splat(ktail), lane])
    jvec = plsc.load_gather(jl16, [_splat(ktail), lane])

    def scalar_at(vec, e):
        return jnp.max(jnp.where(lane == e, vec, -1))

    for e in range(15):
        @pl.when(e < rem)
        def _start():
            j_e = scalar_at(jvec, e)
            t_e = scalar_at(tvec, e)
            pltpu.make_async_copy(g_hbm.at[pl.ds(j_e, 1)],
                                  bufG.at[pl.ds(e, 1)], semG).start()
            pltpu.make_async_copy(h_ref.at[pl.ds(t_e, 1)],
                                  bufH.at[pl.ds(e, 1)], semH).start()

    for e in range(15):
        @pl.when(e < rem)
        def _apply():
            pltpu.make_async_copy(g_hbm.at[pl.ds(0, 1)],
                                  bufG.at[pl.ds(e, 1)], semG).wait()
            pltpu.make_async_copy(g_hbm.at[pl.ds(0, 1)],
                                  bufH.at[pl.ds(e, 1)], semH).wait()
            for c in range(8):
                s = pl.ds(c * L, L)
                bufH[e, s] = bufH[e, s] + bufG[e, s]
            t_e = scalar_at(tvec, e)
            pltpu.make_async_copy(bufH.at[pl.ds(e, 1)],
                                  h_ref.at[pl.ds(t_e, 1)], semS).start()

    for e in range(15):
        @pl.when(e < rem)
        def _drain():
            pltpu.make_async_copy(bufH.at[pl.ds(e, 1)],
                                  h_ref.at[pl.ds(0, 1)], semS).wait()


def _rmw(lists, g, h_ref, tp):
    tpb = tp // NW
    ch128 = (tpb + 127) // 128
    ch16 = tpb // L
    body = functools.partial(_rmw_body, ch128=ch128, ch16=ch16)
    i32 = jnp.int32
    pl.kernel(
        body,
        out_type=(),
        mesh=plsc.VectorSubcoreMesh(**_MESH),
        scratch_types=[
            pltpu.VMEM((ch128, 128), i32),
            pltpu.VMEM((ch128, 128), i32),
            pltpu.VMEM((ch16, L), i32),
            pltpu.VMEM((ch16, L), i32),
            pltpu.VMEM((L,), i32),
            pltpu.VMEM((128, D), jnp.float32),
            pltpu.VMEM((128, D), jnp.float32),
            pltpu.SemaphoreType.DMA,
            pltpu.SemaphoreType.DMA,
            pltpu.SemaphoreType.DMA,
        ],
        compiler_params=pltpu.CompilerParams(needs_layout_passes=False, use_tc_tiling_on_sc=True),
    )(*lists, g, h_ref)


def _prep_body(w0, b0, bup0, w0b, w1, b1, bup1, w1b,
               w0e, b0e, w1e, b1e):
    f32 = jnp.float32
    w0e[...] = w0[...] + jnp.dot(w0[...], w0b[...], preferred_element_type=f32)
    b0e[...] = b0[...] + jnp.dot(b0[...], w0b[...], preferred_element_type=f32) + bup0[...]
    w1e[...] = w1[...] + jnp.dot(w1[...], w1b[...], preferred_element_type=f32)
    b1e[...] = b1[...] + jnp.dot(b1[...], w1b[...], preferred_element_type=f32) + bup1[...]


def _prep(W0, b0r, bup0r, Wup0b, W1, b1r, bup1r, Wup1b):
    f32 = jnp.float32
    return pl.pallas_call(
        _prep_body,
        out_shape=(
            jax.ShapeDtypeStruct((D, D), f32),
            jax.ShapeDtypeStruct((1, D), f32),
            jax.ShapeDtypeStruct((D, D), f32),
            jax.ShapeDtypeStruct((1, D), f32),
        ),
    )(W0, b0r, bup0r, Wup0b, W1, b1r, bup1r, Wup1b)


def _tc2_body(hn2_ref, w2_ref, b2_ref, wt_ref, h2_ref, g2_ref):
    g = pl.program_id(0)
    rows = g * BLK + lax.broadcasted_iota(jnp.int32, (BLK, 1), 0)
    h = jnp.dot(hn2_ref[...], w2_ref[...],
                preferred_element_type=jnp.float32) + b2_ref[...]
    h2_ref[...] = h
    gv = jnp.dot(h, wt_ref[...], preferred_element_type=jnp.float32)
    g2_ref[...] = jnp.where(rows < N2, gv, 0.0)


def _tc2(hn2, W2, b2r, Wup1t):
    nb = N2P // BLK
    return pl.pallas_call(
        _tc2_body,
        grid=(nb,),
        in_specs=[
            pl.BlockSpec((BLK, D), lambda i: (i, 0)),
            pl.BlockSpec((D, D), lambda i: (0, 0)),
            pl.BlockSpec((1, D), lambda i: (0, 0)),
            pl.BlockSpec((D, D), lambda i: (0, 0)),
        ],
        out_specs=(
            pl.BlockSpec((BLK, D), lambda i: (i, 0)),
            pl.BlockSpec((BLK, D), lambda i: (i, 0)),
        ),
        out_shape=(
            jax.ShapeDtypeStruct((N2, D), jnp.float32),
            jax.ShapeDtypeStruct((N2P, D), jnp.float32),
        ),
    )(hn2, W2, b2r, Wup1t)


def _dense_body(hn_ref, w_ref, b_ref, out_ref):
    out_ref[...] = jnp.dot(hn_ref[...], w_ref[...],
                           preferred_element_type=jnp.float32) + b_ref[...]


def _dense(hn, We, be, nrows):
    return pl.pallas_call(
        _dense_body,
        grid=(nrows // BLK,),
        in_specs=[
            pl.BlockSpec((BLK, D), lambda i: (i, 0)),
            pl.BlockSpec((D, D), lambda i: (0, 0)),
            pl.BlockSpec((1, D), lambda i: (0, 0)),
        ],
        out_specs=pl.BlockSpec((BLK, D), lambda i: (i, 0)),
        out_shape=jax.ShapeDtypeStruct((nrows, D), jnp.float32),
    )(hn, We, be)


def _g1_body(h1_ref, wt_ref, g1_ref):
    g = pl.program_id(0)
    rows = g * BLK + lax.broadcasted_iota(jnp.int32, (BLK, 1), 0)
    gv = jnp.dot(h1_ref[...], wt_ref[...], preferred_element_type=jnp.float32)
    g1_ref[...] = jnp.where(rows < N1, gv, 0.0)


def _g1(h1, Wup0t):
    nb = N1P // BLK
    return pl.pallas_call(
        _g1_body,
        grid=(nb,),
        in_specs=[
            pl.BlockSpec((BLK, D), lambda i: (jnp.minimum(i, 24), 0)),
            pl.BlockSpec((D, D), lambda i: (0, 0)),
        ],
        out_specs=pl.BlockSpec((BLK, D), lambda i: (i, 0)),
        out_shape=jax.ShapeDtypeStruct((N1P, D), jnp.float32),
    )(h1, Wup0t)


def kernel(hn0, hn1, hn2, idx1, idx2, W0, b0, W1, b1, W2, b2, Wup0, bup0,
           Wup1, bup1):
    b0r, b1r, b2r = b0.reshape(1, D), b1.reshape(1, D), b2.reshape(1, D)
    bup0r, bup1r = bup0.reshape(1, D), bup1.reshape(1, D)
    Wup0t, Wup0b = Wup0[:D], Wup0[D:]
    Wup1t, Wup1b = Wup1[:D], Wup1[D:]

    W0e, b0e, W1e, b1e = _prep(W0, b0r, bup0r, Wup0b, W1, b1r, bup1r, Wup1b)

    lists2 = _invert(idx2, N2, T1P, zbase=N2)
    lists1 = _invert(idx1, N1, T0P, zbase=N1)

    h2, g2pad = _tc2(hn2, W2, b2r, Wup1t)

    h1d = _dense(hn1, W1e, b1e, N1)
    h1_ref = jax.new_ref(h1d)
    _rmw(lists2, g2pad, h1_ref, T1P)
    h1 = jax.freeze(h1_ref)

    g1pad = _g1(h1, Wup0t)

    h0d = _dense(hn0, W0e, b0e, N0)
    h0_ref = jax.new_ref(h0d)
    _rmw(lists1, g1pad, h0_ref, T0P)
    h0 = jax.freeze(h0_ref)

    return (h0, h1, h2)


# speedup vs baseline: 1.0154x; 1.0132x over previous
import functools

import jax
import jax.numpy as jnp
from jax import lax
from jax.experimental import pallas as pl
from jax.experimental.pallas import tpu as pltpu
from jax.experimental.pallas import tpu_sc as plsc

N0, N1, N2, D = 100000, 25000, 6250, 128
L = 16
NW = 32
BLK = 1000

N2P, N1P = 7000, 26000
T1P = 25088
T0P = 100352
SENT = 1 << 26

_MESH = dict(core_axis_name="c", subcore_axis_name="s")


def _wid():
    return lax.axis_index("s") * 2 + lax.axis_index("c")


def _lane():
    return lax.broadcasted_iota(jnp.int32, (L,), 0)


def _splat(x):
    return jnp.broadcast_to(x, (L,)).astype(jnp.int32)


def _invert_body(idx_hbm, tl128_hbm, jl128_hbm, tl16_hbm, jl16_hbm, cnt_hbm,
                 idx_buf, jw_buf, tl128, jl128, tl16, jl16, cntbuf,
                 *, n, tpb, ch128, ch16, zbase):
    wid = _wid()
    lo = wid * tpb
    lane = _lane()
    zvec = zbase + wid * L + lane

    pltpu.sync_copy(idx_hbm, idx_buf.at[pl.ds(0, n)])
    nfloor = n - n % L
    tail = idx_buf[pl.ds(nfloor, L)]
    idx_buf[pl.ds(nfloor, L)] = jnp.where(lane < n - nfloor, tail, SENT)
    idx_buf[pl.ds(nfloor + L, L)] = jnp.full((L,), SENT, jnp.int32)

    minus1 = jnp.full((L,), -1, jnp.int32)

    def init_jw(i, _):
        jw_buf[pl.ds(i * L, L)] = minus1
        return 0

    lax.fori_loop(0, tpb // L, init_jw, 0)

    def init_lists(i, _):
        for c in range(8):
            tl128[i, pl.ds(c * L, L)] = jnp.zeros((L,), jnp.int32)
            jl128[i, pl.ds(c * L, L)] = zvec
        return 0

    lax.fori_loop(0, ch128, init_lists, 0)

    def init_lists16(i, _):
        tl16[i, :] = jnp.zeros((L,), jnp.int32)
        jl16[i, :] = zvec
        return 0

    lax.fori_loop(0, ch16, init_lists16, 0)

    perm = jnp.minimum(lane + 1, L - 1)

    def scan_one(cc):
        v = idx_buf[pl.ds(cc * L, L)]
        key = v * L + lane
        skey = plsc.sort_key_val(key, key)[0]
        tgt = skey >> 4
        nxt = skey.at[perm].get(mode="promise_in_bounds")
        keep = (tgt != (nxt >> 4)) | (lane == L - 1)
        m = keep & (tgt >= lo) & (tgt < lo + tpb)
        jval = cc * L + (skey & (L - 1))
        return tgt - lo, jval, m

    def scan_pair(c, _):
        i0, j0, m0 = scan_one(2 * c)
        i1, j1, m1 = scan_one(2 * c + 1)
        plsc.store_scatter(jw_buf, [i0], j0, mask=m0)
        plsc.store_scatter(jw_buf, [i1], j1, mask=m1)
        return 0

    lax.fori_loop(0, (n + 2 * L - 1) // (2 * L), scan_pair, 0)

    def comp_body(i, off):
        v = jw_buf[pl.ds(i * L, L)]
        m = v >= 0
        mi = m.astype(jnp.int32)
        pos = off + plsc.cumsum(mi) - 1
        t = lo + i * L + lane
        plsc.store_scatter(tl128, [pos >> 7, pos & 127], t, mask=m)
        plsc.store_scatter(jl128, [pos >> 7, pos & 127], v, mask=m)
        plsc.store_scatter(tl16, [pos >> 4, pos & 15], t, mask=m)
        plsc.store_scatter(jl16, [pos >> 4, pos & 15], v, mask=m)
        return off + jnp.sum(mi)

    cnt = lax.fori_loop(0, tpb // L, comp_body, jnp.int32(0))

    cntbuf[...] = _splat(cnt)
    pltpu.sync_copy(tl128, tl128_hbm.at[wid])
    pltpu.sync_copy(jl128, jl128_hbm.at[wid])
    pltpu.sync_copy(tl16, tl16_hbm.at[wid])
    pltpu.sync_copy(jl16, jl16_hbm.at[wid])
    pltpu.sync_copy(cntbuf, cnt_hbm.at[wid])


def _invert(idx, n, tp, zbase):
    tpb = tp // NW
    ch128 = (tpb + 127) // 128
    ch16 = tpb // L
    body = functools.partial(_invert_body, n=n, tpb=tpb, ch128=ch128,
                             ch16=ch16, zbase=zbase)
    i32 = jnp.int32
    return pl.kernel(
        body,
        out_type=(
            jax.ShapeDtypeStruct((NW, ch128, 128), i32),
            jax.ShapeDtypeStruct((NW, ch128, 128), i32),
            jax.ShapeDtypeStruct((NW, ch16, L), i32),
            jax.ShapeDtypeStruct((NW, ch16, L), i32),
            jax.ShapeDtypeStruct((NW, L), i32),
        ),
        mesh=plsc.VectorSubcoreMesh(**_MESH),
        scratch_types=[
            pltpu.VMEM((25024,), i32),
            pltpu.VMEM((tpb,), i32),
            pltpu.VMEM((ch128, 128), i32),
            pltpu.VMEM((ch128, 128), i32),
            pltpu.VMEM((ch16, L), i32),
            pltpu.VMEM((ch16, L), i32),
            pltpu.VMEM((L,), i32),
        ],
        compiler_params=pltpu.CompilerParams(needs_layout_passes=False, use_tc_tiling_on_sc=True),
    )(idx)


def _rmw_body(tl128_hbm, jl128_hbm, tl16_hbm, jl16_hbm, cnt_hbm, g_hbm, h_ref,
              tl128, jl128, tl16, jl16, cntbuf, bufG, bufH, bufG2, bufH2,
              semG, semH, semS, semG2, semH2, semS2, *, ch128, ch16):
    wid = _wid()
    lane = _lane()
    pltpu.sync_copy(tl128_hbm.at[wid], tl128)
    pltpu.sync_copy(jl128_hbm.at[wid], jl128)
    pltpu.sync_copy(tl16_hbm.at[wid], tl16)
    pltpu.sync_copy(jl16_hbm.at[wid], jl16)
    pltpu.sync_copy(cnt_hbm.at[wid], cntbuf)
    cnt = jnp.max(cntbuf[...])
    n128 = cnt >> 7
    n16 = (cnt & 127) >> 4
    rem = cnt & 15

    def add_rows(bh, bg, nrows):
        def row_body(r, _):
            for c in range(8):
                s = pl.ds(c * L, L)
                bh[r, s] = bh[r, s] + bg[r, s]
            return 0
        lax.fori_loop(0, nrows, row_body, 0)

    def big_pair(p, _):
        k0 = 2 * p
        k1 = 2 * p + 1
        gh0 = pltpu.async_copy(g_hbm.at[jl128.at[k0]], bufG, semG)
        hh0 = pltpu.async_copy(h_ref.at[tl128.at[k0]], bufH, semH)
        gh1 = pltpu.async_copy(g_hbm.at[jl128.at[k1]], bufG2, semG2)
        hh1 = pltpu.async_copy(h_ref.at[tl128.at[k1]], bufH2, semH2)
        gh0.wait()
        hh0.wait()
        add_rows(bufH, bufG, 128)
        sc0 = pltpu.async_copy(bufH, h_ref.at[tl128.at[k0]], semS)
        gh1.wait()
        hh1.wait()
        add_rows(bufH2, bufG2, 128)
        sc1 = pltpu.async_copy(bufH2, h_ref.at[tl128.at[k1]], semS2)
        sc0.wait()
        sc1.wait()
        return 0

    lax.fori_loop(0, n128 >> 1, big_pair, 0)

    def big_last(k, _):
        gh = pltpu.async_copy(g_hbm.at[jl128.at[k]], bufG, semG)
        hh = pltpu.async_copy(h_ref.at[tl128.at[k]], bufH, semH)
        gh.wait()
        hh.wait()
        add_rows(bufH, bufG, 128)
        pltpu.async_copy(bufH, h_ref.at[tl128.at[k]], semS).wait()
        return 0

    lax.fori_loop(n128 & ~1, n128, big_last, 0)

    def mid_chunk(i, _):
        k = n128 * 8 + i
        gh = pltpu.async_copy(g_hbm.at[jl16.at[k]], bufG.at[pl.ds(0, L)],
                              semG)
        hh = pltpu.async_copy(h_ref.at[tl16.at[k]], bufH.at[pl.ds(0, L)],
                              semH)
        gh.wait()
        hh.wait()
        add_rows(bufH, bufG, L)
        pltpu.async_copy(bufH.at[pl.ds(0, L)], h_ref.at[tl16.at[k]],
                         semS).wait()
        return 0

    lax.fori_loop(0, n16, mid_chunk, 0)

    ktail = n128 * 8 + n16
    tvec = plsc.load_gather(tl16, [_splat(ktail), lane])
    jvec = plsc.load_gather(jl16, [_splat(ktail), lane])

    def scalar_at(vec, e):
        return jnp.max(jnp.where(lane == e, vec, -1))

    for e in range(15):
        @pl.when(e < rem)
        def _start():
            j_e = scalar_at(jvec, e)
            t_e = scalar_at(tvec, e)
            pltpu.make_async_copy(g_hbm.at[pl.ds(j_e, 1)],
                                  bufG.at[pl.ds(e, 1)], semG).start()
            pltpu.make_async_copy(h_ref.at[pl.ds(t_e, 1)],
                                  bufH.at[pl.ds(e, 1)], semH).start()

    for e in range(15):
        @pl.when(e < rem)
        def _apply():
            pltpu.make_async_copy(g_hbm.at[pl.ds(0, 1)],
                                  bufG.at[pl.ds(e, 1)], semG).wait()
            pltpu.make_async_copy(g_hbm.at[pl.ds(0, 1)],
                                  bufH.at[pl.ds(e, 1)], semH).wait()
            for c in range(8):
                s = pl.ds(c * L, L)
                bufH[e, s] = bufH[e, s] + bufG[e, s]
            t_e = scalar_at(tvec, e)
            pltpu.make_async_copy(bufH.at[pl.ds(e, 1)],
                                  h_ref.at[pl.ds(t_e, 1)], semS).start()

    for e in range(15):
        @pl.when(e < rem)
        def _drain():
            pltpu.make_async_copy(bufH.at[pl.ds(e, 1)],
                                  h_ref.at[pl.ds(0, 1)], semS).wait()


def _rmw(lists, g, h_ref, tp):
    tpb = tp // NW
    ch128 = (tpb + 127) // 128
    ch16 = tpb // L
    body = functools.partial(_rmw_body, ch128=ch128, ch16=ch16)
    i32 = jnp.int32
    pl.kernel(
        body,
        out_type=(),
        mesh=plsc.VectorSubcoreMesh(**_MESH),
        scratch_types=[
            pltpu.VMEM((ch128, 128), i32),
            pltpu.VMEM((ch128, 128), i32),
            pltpu.VMEM((ch16, L), i32),
            pltpu.VMEM((ch16, L), i32),
            pltpu.VMEM((L,), i32),
            pltpu.VMEM((128, D), jnp.float32),
            pltpu.VMEM((128, D), jnp.float32),
            pltpu.VMEM((128, D), jnp.float32),
            pltpu.VMEM((128, D), jnp.float32),
            pltpu.SemaphoreType.DMA,
            pltpu.SemaphoreType.DMA,
            pltpu.SemaphoreType.DMA,
            pltpu.SemaphoreType.DMA,
            pltpu.SemaphoreType.DMA,
            pltpu.SemaphoreType.DMA,
        ],
        compiler_params=pltpu.CompilerParams(needs_layout_passes=False, use_tc_tiling_on_sc=True),
    )(*lists, g, h_ref)


def _prep_body(w0, b0, bup0, w0b, w1, b1, bup1, w1b,
               w0e, b0e, w1e, b1e):
    f32 = jnp.float32
    w0e[...] = w0[...] + jnp.dot(w0[...], w0b[...], preferred_element_type=f32)
    b0e[...] = b0[...] + jnp.dot(b0[...], w0b[...], preferred_element_type=f32) + bup0[...]
    w1e[...] = w1[...] + jnp.dot(w1[...], w1b[...], preferred_element_type=f32)
    b1e[...] = b1[...] + jnp.dot(b1[...], w1b[...], preferred_element_type=f32) + bup1[...]


def _prep(W0, b0r, bup0r, Wup0b, W1, b1r, bup1r, Wup1b):
    f32 = jnp.float32
    return pl.pallas_call(
        _prep_body,
        out_shape=(
            jax.ShapeDtypeStruct((D, D), f32),
            jax.ShapeDtypeStruct((1, D), f32),
            jax.ShapeDtypeStruct((D, D), f32),
            jax.ShapeDtypeStruct((1, D), f32),
        ),
    )(W0, b0r, bup0r, Wup0b, W1, b1r, bup1r, Wup1b)


def _tc2_body(hn2_ref, w2_ref, b2_ref, wt_ref, h2_ref, g2_ref):
    g = pl.program_id(0)
    rows = g * BLK + lax.broadcasted_iota(jnp.int32, (BLK, 1), 0)
    h = jnp.dot(hn2_ref[...], w2_ref[...],
                preferred_element_type=jnp.float32) + b2_ref[...]
    h2_ref[...] = h
    gv = jnp.dot(h, wt_ref[...], preferred_element_type=jnp.float32)
    g2_ref[...] = jnp.where(rows < N2, gv, 0.0)


def _tc2(hn2, W2, b2r, Wup1t):
    nb = N2P // BLK
    return pl.pallas_call(
        _tc2_body,
        grid=(nb,),
        in_specs=[
            pl.BlockSpec((BLK, D), lambda i: (i, 0)),
            pl.BlockSpec((D, D), lambda i: (0, 0)),
            pl.BlockSpec((1, D), lambda i: (0, 0)),
            pl.BlockSpec((D, D), lambda i: (0, 0)),
        ],
        out_specs=(
            pl.BlockSpec((BLK, D), lambda i: (i, 0)),
            pl.BlockSpec((BLK, D), lambda i: (i, 0)),
        ),
        out_shape=(
            jax.ShapeDtypeStruct((N2, D), jnp.float32),
            jax.ShapeDtypeStruct((N2P, D), jnp.float32),
        ),
    )(hn2, W2, b2r, Wup1t)


def _dense_body(hn_ref, w_ref, b_ref, out_ref):
    out_ref[...] = jnp.dot(hn_ref[...], w_ref[...],
                           preferred_element_type=jnp.float32) + b_ref[...]


def _dense(hn, We, be, nrows):
    return pl.pallas_call(
        _dense_body,
        grid=(nrows // BLK,),
        in_specs=[
            pl.BlockSpec((BLK, D), lambda i: (i, 0)),
            pl.BlockSpec((D, D), lambda i: (0, 0)),
            pl.BlockSpec((1, D), lambda i: (0, 0)),
        ],
        out_specs=pl.BlockSpec((BLK, D), lambda i: (i, 0)),
        out_shape=jax.ShapeDtypeStruct((nrows, D), jnp.float32),
    )(hn, We, be)


def _g1_body(h1_ref, wt_ref, g1_ref):
    g = pl.program_id(0)
    rows = g * BLK + lax.broadcasted_iota(jnp.int32, (BLK, 1), 0)
    gv = jnp.dot(h1_ref[...], wt_ref[...], preferred_element_type=jnp.float32)
    g1_ref[...] = jnp.where(rows < N1, gv, 0.0)


def _g1(h1, Wup0t):
    nb = N1P // BLK
    return pl.pallas_call(
        _g1_body,
        grid=(nb,),
        in_specs=[
            pl.BlockSpec((BLK, D), lambda i: (jnp.minimum(i, 24), 0)),
            pl.BlockSpec((D, D), lambda i: (0, 0)),
        ],
        out_specs=pl.BlockSpec((BLK, D), lambda i: (i, 0)),
        out_shape=jax.ShapeDtypeStruct((N1P, D), jnp.float32),
    )(h1, Wup0t)


def kernel(hn0, hn1, hn2, idx1, idx2, W0, b0, W1, b1, W2, b2, Wup0, bup0,
           Wup1, bup1):
    b0r, b1r, b2r = b0.reshape(1, D), b1.reshape(1, D), b2.reshape(1, D)
    bup0r, bup1r = bup0.reshape(1, D), bup1.reshape(1, D)
    Wup0t, Wup0b = Wup0[:D], Wup0[D:]
    Wup1t, Wup1b = Wup1[:D], Wup1[D:]

    W0e, b0e, W1e, b1e = _prep(W0, b0r, bup0r, Wup0b, W1, b1r, bup1r, Wup1b)

    lists2 = _invert(idx2, N2, T1P, zbase=N2)
    lists1 = _invert(idx1, N1, T0P, zbase=N1)

    h2, g2pad = _tc2(hn2, W2, b2r, Wup1t)
    h1d = _dense(hn1, W1e, b1e, N1)
    h0d = _dense(hn0, W0e, b0e, N0)

    h1_ref = jax.new_ref(h1d)
    _rmw(lists2, g2pad, h1_ref, T1P)
    h1 = jax.freeze(h1_ref)

    g1pad = _g1(h1, Wup0t)

    h0_ref = jax.new_ref(h0d)
    _rmw(lists1, g1pad, h0_ref, T0P)
    h0 = jax.freeze(h0_ref)

    return (h0, h1, h2)


# speedup vs baseline: 1.0926x; 1.0760x over previous
import functools

import jax
import jax.numpy as jnp
from jax import lax
from jax.experimental import pallas as pl
from jax.experimental.pallas import tpu as pltpu
from jax.experimental.pallas import tpu_sc as plsc

N0, N1, N2, D = 100000, 25000, 6250, 128
L = 16
NW = 32
BLK = 1000

N2P, N1P = 7000, 26000
T1P = 25088
T0P = 100352
SENT = 1 << 26

_MESH = dict(core_axis_name="c", subcore_axis_name="s")


def _wid():
    return lax.axis_index("s") * 2 + lax.axis_index("c")


def _lane():
    return lax.broadcasted_iota(jnp.int32, (L,), 0)


def _splat(x):
    return jnp.broadcast_to(x, (L,)).astype(jnp.int32)


def _invert_body(idx_hbm, tl128_hbm, jl128_hbm, tl16_hbm, jl16_hbm, cnt_hbm,
                 idx_buf, jw_buf, tl128, jl128, tl16, jl16, cntbuf,
                 *, n, tpb, ch128, ch16, zbase):
    wid = _wid()
    lo = wid * tpb
    lane = _lane()
    zvec = zbase + wid * L + lane

    pltpu.sync_copy(idx_hbm, idx_buf.at[pl.ds(0, n)])
    nfloor = n - n % L
    tail = idx_buf[pl.ds(nfloor, L)]
    idx_buf[pl.ds(nfloor, L)] = jnp.where(lane < n - nfloor, tail, SENT)
    idx_buf[pl.ds(nfloor + L, L)] = jnp.full((L,), SENT, jnp.int32)

    minus1 = jnp.full((L,), -1, jnp.int32)

    def init_jw(i, _):
        jw_buf[pl.ds(i * L, L)] = minus1
        return 0

    lax.fori_loop(0, tpb // L, init_jw, 0)

    def init_lists(i, _):
        for c in range(8):
            tl128[i, pl.ds(c * L, L)] = jnp.zeros((L,), jnp.int32)
            jl128[i, pl.ds(c * L, L)] = zvec
        return 0

    lax.fori_loop(0, ch128, init_lists, 0)

    def init_lists16(i, _):
        tl16[i, :] = jnp.zeros((L,), jnp.int32)
        jl16[i, :] = zvec
        return 0

    lax.fori_loop(0, ch16, init_lists16, 0)

    perm = jnp.minimum(lane + 1, L - 1)

    def scan_one(cc):
        v = idx_buf[pl.ds(cc * L, L)]
        key = v * L + lane
        skey = plsc.sort_key_val(key, key)[0]
        tgt = skey >> 4
        nxt = skey.at[perm].get(mode="promise_in_bounds")
        keep = (tgt != (nxt >> 4)) | (lane == L - 1)
        m = keep & (tgt >= lo) & (tgt < lo + tpb)
        jval = cc * L + (skey & (L - 1))
        return tgt - lo, jval, m

    def scan_pair(c, _):
        i0, j0, m0 = scan_one(2 * c)
        i1, j1, m1 = scan_one(2 * c + 1)
        plsc.store_scatter(jw_buf, [i0], j0, mask=m0)
        plsc.store_scatter(jw_buf, [i1], j1, mask=m1)
        return 0

    lax.fori_loop(0, (n + 2 * L - 1) // (2 * L), scan_pair, 0)

    def comp_body(i, off):
        v = jw_buf[pl.ds(i * L, L)]
        m = v >= 0
        mi = m.astype(jnp.int32)
        pos = off + plsc.cumsum(mi) - 1
        t = lo + i * L + lane
        plsc.store_scatter(tl128, [pos >> 7, pos & 127], t, mask=m)
        plsc.store_scatter(jl128, [pos >> 7, pos & 127], v, mask=m)
        plsc.store_scatter(tl16, [pos >> 4, pos & 15], t, mask=m)
        plsc.store_scatter(jl16, [pos >> 4, pos & 15], v, mask=m)
        return off + jnp.sum(mi)

    cnt = lax.fori_loop(0, tpb // L, comp_body, jnp.int32(0))

    cntbuf[...] = _splat(cnt)
    pltpu.sync_copy(tl128, tl128_hbm.at[wid])
    pltpu.sync_copy(jl128, jl128_hbm.at[wid])
    pltpu.sync_copy(tl16, tl16_hbm.at[wid])
    pltpu.sync_copy(jl16, jl16_hbm.at[wid])
    pltpu.sync_copy(cntbuf, cnt_hbm.at[wid])


def _invert(idx, n, tp, zbase):
    tpb = tp // NW
    ch128 = (tpb + 127) // 128
    ch16 = tpb // L
    body = functools.partial(_invert_body, n=n, tpb=tpb, ch128=ch128,
                             ch16=ch16, zbase=zbase)
    i32 = jnp.int32
    return pl.kernel(
        body,
        out_type=(
            jax.ShapeDtypeStruct((NW, ch128, 128), i32),
            jax.ShapeDtypeStruct((NW, ch128, 128), i32),
            jax.ShapeDtypeStruct((NW, ch16, L), i32),
            jax.ShapeDtypeStruct((NW, ch16, L), i32),
            jax.ShapeDtypeStruct((NW, L), i32),
        ),
        mesh=plsc.VectorSubcoreMesh(**_MESH),
        scratch_types=[
            pltpu.VMEM((25024,), i32),
            pltpu.VMEM((tpb,), i32),
            pltpu.VMEM((ch128, 128), i32),
            pltpu.VMEM((ch128, 128), i32),
            pltpu.VMEM((ch16, L), i32),
            pltpu.VMEM((ch16, L), i32),
            pltpu.VMEM((L,), i32),
        ],
        compiler_params=pltpu.CompilerParams(needs_layout_passes=False, use_tc_tiling_on_sc=True),
    )(idx)


def _rmw_body(tl128_hbm, jl128_hbm, tl16_hbm, jl16_hbm, cnt_hbm, *gh_args,
              ch128, ch16, npairs):
    gh_pairs = [(gh_args[2 * i], gh_args[2 * i + 1]) for i in range(npairs)]
    (tl128, jl128, tl16, jl16, cntbuf, bufG, bufH, bufG2, bufH2,
     semG, semH, semS, semG2, semH2, semS2) = gh_args[2 * npairs:]
    wid = _wid()
    lane = _lane()
    pltpu.sync_copy(tl128_hbm.at[wid], tl128)
    pltpu.sync_copy(jl128_hbm.at[wid], jl128)
    pltpu.sync_copy(tl16_hbm.at[wid], tl16)
    pltpu.sync_copy(jl16_hbm.at[wid], jl16)
    pltpu.sync_copy(cnt_hbm.at[wid], cntbuf)
    cnt = jnp.max(cntbuf[...])
    n128 = cnt >> 7
    n16 = (cnt & 127) >> 4
    rem = cnt & 15

    def add_rows(bh, bg, nrows):
        def row_body(r, _):
            for c in range(8):
                s = pl.ds(c * L, L)
                bh[r, s] = bh[r, s] + bg[r, s]
            return 0
        lax.fori_loop(0, nrows, row_body, 0)

    for g_hbm, h_ref in gh_pairs:
        _rmw_one(g_hbm, h_ref, tl128, jl128, tl16, jl16, lane, n128, n16, rem,
                 bufG, bufH, bufG2, bufH2, semG, semH, semS, semG2, semH2,
                 semS2, add_rows)


def _rmw_one(g_hbm, h_ref, tl128, jl128, tl16, jl16, lane, n128, n16, rem,
             bufG, bufH, bufG2, bufH2, semG, semH, semS, semG2, semH2, semS2,
             add_rows):
    def big_pair(p, _):
        k0 = 2 * p
        k1 = 2 * p + 1
        gh0 = pltpu.async_copy(g_hbm.at[jl128.at[k0]], bufG, semG)
        hh0 = pltpu.async_copy(h_ref.at[tl128.at[k0]], bufH, semH)
        gh1 = pltpu.async_copy(g_hbm.at[jl128.at[k1]], bufG2, semG2)
        hh1 = pltpu.async_copy(h_ref.at[tl128.at[k1]], bufH2, semH2)
        gh0.wait()
        hh0.wait()
        add_rows(bufH, bufG, 128)
        sc0 = pltpu.async_copy(bufH, h_ref.at[tl128.at[k0]], semS)
        gh1.wait()
        hh1.wait()
        add_rows(bufH2, bufG2, 128)
        sc1 = pltpu.async_copy(bufH2, h_ref.at[tl128.at[k1]], semS2)
        sc0.wait()
        sc1.wait()
        return 0

    lax.fori_loop(0, n128 >> 1, big_pair, 0)

    def big_last(k, _):
        gh = pltpu.async_copy(g_hbm.at[jl128.at[k]], bufG, semG)
        hh = pltpu.async_copy(h_ref.at[tl128.at[k]], bufH, semH)
        gh.wait()
        hh.wait()
        add_rows(bufH, bufG, 128)
        pltpu.async_copy(bufH, h_ref.at[tl128.at[k]], semS).wait()
        return 0

    lax.fori_loop(n128 & ~1, n128, big_last, 0)

    def mid_chunk(i, _):
        k = n128 * 8 + i
        gh = pltpu.async_copy(g_hbm.at[jl16.at[k]], bufG.at[pl.ds(0, L)],
                              semG)
        hh = pltpu.async_copy(h_ref.at[tl16.at[k]], bufH.at[pl.ds(0, L)],
                              semH)
        gh.wait()
        hh.wait()
        add_rows(bufH, bufG, L)
        pltpu.async_copy(bufH.at[pl.ds(0, L)], h_ref.at[tl16.at[k]],
                         semS).wait()
        return 0

    lax.fori_loop(0, n16, mid_chunk, 0)

    ktail = n128 * 8 + n16
    tvec = plsc.load_gather(tl16, [_splat(ktail), lane])
    jvec = plsc.load_gather(jl16, [_splat(ktail), lane])

    def scalar_at(vec, e):
        return jnp.max(jnp.where(lane == e, vec, -1))

    for e in range(15):
        @pl.when(e < rem)
        def _start():
            j_e = scalar_at(jvec, e)
            t_e = scalar_at(tvec, e)
            pltpu.make_async_copy(g_hbm.at[pl.ds(j_e, 1)],
                                  bufG.at[pl.ds(e, 1)], semG).start()
            pltpu.make_async_copy(h_ref.at[pl.ds(t_e, 1)],
                                  bufH.at[pl.ds(e, 1)], semH).start()

    for e in range(15):
        @pl.when(e < rem)
        def _apply():
            pltpu.make_async_copy(g_hbm.at[pl.ds(0, 1)],
                                  bufG.at[pl.ds(e, 1)], semG).wait()
            pltpu.make_async_copy(g_hbm.at[pl.ds(0, 1)],
                                  bufH.at[pl.ds(e, 1)], semH).wait()
            for c in range(8):
                s = pl.ds(c * L, L)
                bufH[e, s] = bufH[e, s] + bufG[e, s]
            t_e = scalar_at(tvec, e)
            pltpu.make_async_copy(bufH.at[pl.ds(e, 1)],
                                  h_ref.at[pl.ds(t_e, 1)], semS).start()

    for e in range(15):
        @pl.when(e < rem)
        def _drain():
            pltpu.make_async_copy(bufH.at[pl.ds(e, 1)],
                                  h_ref.at[pl.ds(0, 1)], semS).wait()


def _rmw(lists, gh_pairs, tp):
    tpb = tp // NW
    ch128 = (tpb + 127) // 128
    ch16 = tpb // L
    body = functools.partial(_rmw_body, ch128=ch128, ch16=ch16,
                             npairs=len(gh_pairs))
    i32 = jnp.int32
    flat = [x for gh in gh_pairs for x in gh]
    pl.kernel(
        body,
        out_type=(),
        mesh=plsc.VectorSubcoreMesh(**_MESH),
        scratch_types=[
            pltpu.VMEM((ch128, 128), i32),
            pltpu.VMEM((ch128, 128), i32),
            pltpu.VMEM((ch16, L), i32),
            pltpu.VMEM((ch16, L), i32),
            pltpu.VMEM((L,), i32),
            pltpu.VMEM((128, D), jnp.float32),
            pltpu.VMEM((128, D), jnp.float32),
            pltpu.VMEM((128, D), jnp.float32),
            pltpu.VMEM((128, D), jnp.float32),
            pltpu.SemaphoreType.DMA,
            pltpu.SemaphoreType.DMA,
            pltpu.SemaphoreType.DMA,
            pltpu.SemaphoreType.DMA,
            pltpu.SemaphoreType.DMA,
            pltpu.SemaphoreType.DMA,
        ],
        compiler_params=pltpu.CompilerParams(needs_layout_passes=False, use_tc_tiling_on_sc=True),
    )(*lists, *flat)


def _prep_body(w0, b0, bup0, w0b, w1, b1, bup1, w1b, w1t, w0t,
               w0e, b0e, w1e, b1e, wc):
    f32 = jnp.float32
    w0e[...] = w0[...] + jnp.dot(w0[...], w0b[...], preferred_element_type=f32)
    b0e[...] = b0[...] + jnp.dot(b0[...], w0b[...], preferred_element_type=f32) + bup0[...]
    w1e[...] = w1[...] + jnp.dot(w1[...], w1b[...], preferred_element_type=f32)
    b1e[...] = b1[...] + jnp.dot(b1[...], w1b[...], preferred_element_type=f32) + bup1[...]
    wc[...] = jnp.dot(w1t[...], w0t[...], preferred_element_type=f32)


def _prep(W0, b0r, bup0r, Wup0b, W1, b1r, bup1r, Wup1b, Wup1t, Wup0t):
    f32 = jnp.float32
    return pl.pallas_call(
        _prep_body,
        out_shape=(
            jax.ShapeDtypeStruct((D, D), f32),
            jax.ShapeDtypeStruct((1, D), f32),
            jax.ShapeDtypeStruct((D, D), f32),
            jax.ShapeDtypeStruct((1, D), f32),
            jax.ShapeDtypeStruct((D, D), f32),
        ),
    )(W0, b0r, bup0r, Wup0b, W1, b1r, bup1r, Wup1b, Wup1t, Wup0t)


def _tc2_body(hn2_ref, w2_ref, b2_ref, wt_ref, wc_ref, h2_ref, g2_ref,
              gc2_ref):
    g = pl.program_id(0)
    rows = g * BLK + lax.broadcasted_iota(jnp.int32, (BLK, 1), 0)
    h = jnp.dot(hn2_ref[...], w2_ref[...],
                preferred_element_type=jnp.float32) + b2_ref[...]
    h2_ref[...] = h
    gv = jnp.dot(h, wt_ref[...], preferred_element_type=jnp.float32)
    g2_ref[...] = jnp.where(rows < N2, gv, 0.0)
    gcv = jnp.dot(h, wc_ref[...], preferred_element_type=jnp.float32)
    gc2_ref[...] = jnp.where(rows < N2, gcv, 0.0)


def _tc2(hn2, W2, b2r, Wup1t, Wc):
    nb = N2P // BLK
    return pl.pallas_call(
        _tc2_body,
        grid=(nb,),
        in_specs=[
            pl.BlockSpec((BLK, D), lambda i: (i, 0)),
            pl.BlockSpec((D, D), lambda i: (0, 0)),
            pl.BlockSpec((1, D), lambda i: (0, 0)),
            pl.BlockSpec((D, D), lambda i: (0, 0)),
            pl.BlockSpec((D, D), lambda i: (0, 0)),
        ],
        out_specs=(
            pl.BlockSpec((BLK, D), lambda i: (i, 0)),
            pl.BlockSpec((BLK, D), lambda i: (i, 0)),
            pl.BlockSpec((BLK, D), lambda i: (i, 0)),
        ),
        out_shape=(
            jax.ShapeDtypeStruct((N2, D), jnp.float32),
            jax.ShapeDtypeStruct((N2P, D), jnp.float32),
            jax.ShapeDtypeStruct((N2P, D), jnp.float32),
        ),
    )(hn2, W2, b2r, Wup1t, Wc)


def _dense_body(hn_ref, w_ref, b_ref, out_ref):
    out_ref[...] = jnp.dot(hn_ref[...], w_ref[...],
                           preferred_element_type=jnp.float32) + b_ref[...]


def _dense(hn, We, be, nrows):
    return pl.pallas_call(
        _dense_body,
        grid=(nrows // BLK,),
        in_specs=[
            pl.BlockSpec((BLK, D), lambda i: (i, 0)),
            pl.BlockSpec((D, D), lambda i: (0, 0)),
            pl.BlockSpec((1, D), lambda i: (0, 0)),
        ],
        out_specs=pl.BlockSpec((BLK, D), lambda i: (i, 0)),
        out_shape=jax.ShapeDtypeStruct((nrows, D), jnp.float32),
    )(hn, We, be)


def _dense1g_body(hn_ref, w_ref, b_ref, wt_ref, h1_ref, g1_ref):
    nb = N1P // BLK
    r = (nb - 1) - pl.program_id(0)
    rows = r * BLK + lax.broadcasted_iota(jnp.int32, (BLK, 1), 0)
    h = jnp.dot(hn_ref[...], w_ref[...],
                preferred_element_type=jnp.float32) + b_ref[...]
    h1_ref[...] = h
    gv = jnp.dot(h, wt_ref[...], preferred_element_type=jnp.float32)
    g1_ref[...] = jnp.where(rows < N1, gv, 0.0)


def _dense1g(hn1, We, be, Wup0t):
    nb = N1P // BLK
    rev = nb - 1
    return pl.pallas_call(
        _dense1g_body,
        grid=(nb,),
        in_specs=[
            pl.BlockSpec((BLK, D), lambda i: (jnp.minimum(rev - i, rev - 1), 0)),
            pl.BlockSpec((D, D), lambda i: (0, 0)),
            pl.BlockSpec((1, D), lambda i: (0, 0)),
            pl.BlockSpec((D, D), lambda i: (0, 0)),
        ],
        out_specs=(
            pl.BlockSpec((BLK, D), lambda i: (jnp.minimum(rev - i, rev - 1), 0)),
            pl.BlockSpec((BLK, D), lambda i: (rev - i, 0)),
        ),
        out_shape=(
            jax.ShapeDtypeStruct((N1, D), jnp.float32),
            jax.ShapeDtypeStruct((N1P, D), jnp.float32),
        ),
    )(hn1, We, be, Wup0t)


def kernel(hn0, hn1, hn2, idx1, idx2, W0, b0, W1, b1, W2, b2, Wup0, bup0,
           Wup1, bup1):
    b0r, b1r, b2r = b0.reshape(1, D), b1.reshape(1, D), b2.reshape(1, D)
    bup0r, bup1r = bup0.reshape(1, D), bup1.reshape(1, D)
    Wup0t, Wup0b = Wup0[:D], Wup0[D:]
    Wup1t, Wup1b = Wup1[:D], Wup1[D:]

    W0e, b0e, W1e, b1e, Wc = _prep(W0, b0r, bup0r, Wup0b, W1, b1r, bup1r,
                                   Wup1b, Wup1t, Wup0t)

    lists2 = _invert(idx2, N2, T1P, zbase=N2)
    lists1 = _invert(idx1, N1, T0P, zbase=N1)

    h2, g2pad, gc2pad = _tc2(hn2, W2, b2r, Wup1t, Wc)
    h1d, g1d = _dense1g(hn1, W1e, b1e, Wup0t)
    h0d = _dense(hn0, W0e, b0e, N0)

    h1_ref = jax.new_ref(h1d)
    g1_ref = jax.new_ref(g1d)
    _rmw(lists2, [(g2pad, h1_ref), (gc2pad, g1_ref)], T1P)
    h1 = jax.freeze(h1_ref)
    g1pad = jax.freeze(g1_ref)

    h0_ref = jax.new_ref(h0d)
    _rmw(lists1, [(g1pad, h0_ref)], T0P)
    h0 = jax.freeze(h0_ref)

    return (h0, h1, h2)


# speedup vs baseline: 1.2875x; 1.1784x over previous
import functools

import jax
import jax.numpy as jnp
from jax import lax
from jax.experimental import pallas as pl
from jax.experimental.pallas import tpu as pltpu
from jax.experimental.pallas import tpu_sc as plsc

N0, N1, N2, D = 100000, 25000, 6250, 128
L = 16
NW = 32
BLK = 1000

N2P, N1P = 7000, 26000
T1P = 25088
T0P = 100352
SENT = 1 << 26

_MESH = dict(core_axis_name="c", subcore_axis_name="s")


def _wid():
    return lax.axis_index("s") * 2 + lax.axis_index("c")


def _lane():
    return lax.broadcasted_iota(jnp.int32, (L,), 0)


def _splat(x):
    return jnp.broadcast_to(x, (L,)).astype(jnp.int32)


def _invert_body(idx_hbm, tl128_hbm, jl128_hbm, tl16_hbm, jl16_hbm, cnt_hbm,
                 idx_buf, jw_buf, tl128, jl128, tl16, jl16, cntbuf,
                 *, n, tpb, ch128, ch16, zbase):
    wid = _wid()
    lo = wid * tpb
    lane = _lane()
    zvec = zbase + wid * L + lane

    pltpu.sync_copy(idx_hbm, idx_buf.at[pl.ds(0, n)])
    nfloor = n - n % L
    tail = idx_buf[pl.ds(nfloor, L)]
    idx_buf[pl.ds(nfloor, L)] = jnp.where(lane < n - nfloor, tail, SENT)
    idx_buf[pl.ds(nfloor + L, L)] = jnp.full((L,), SENT, jnp.int32)

    minus1 = jnp.full((L,), -1, jnp.int32)

    def init_jw(i, _):
        jw_buf[pl.ds(i * L, L)] = minus1
        return 0

    lax.fori_loop(0, tpb // L, init_jw, 0)

    def init_lists(i, _):
        for c in range(8):
            tl128[i, pl.ds(c * L, L)] = jnp.zeros((L,), jnp.int32)
            jl128[i, pl.ds(c * L, L)] = zvec
        return 0

    lax.fori_loop(0, ch128, init_lists, 0)

    def init_lists16(i, _):
        tl16[i, :] = jnp.zeros((L,), jnp.int32)
        jl16[i, :] = zvec
        return 0

    lax.fori_loop(0, ch16, init_lists16, 0)

    perm = jnp.minimum(lane + 1, L - 1)

    def scan_one(cc):
        v = idx_buf[pl.ds(cc * L, L)]
        key = v * L + lane
        skey = plsc.sort_key_val(key, key)[0]
        tgt = skey >> 4
        nxt = skey.at[perm].get(mode="promise_in_bounds")
        keep = (tgt != (nxt >> 4)) | (lane == L - 1)
        m = keep & (tgt >= lo) & (tgt < lo + tpb)
        jval = cc * L + (skey & (L - 1))
        return tgt - lo, jval, m

    def scan_pair(c, _):
        i0, j0, m0 = scan_one(2 * c)
        i1, j1, m1 = scan_one(2 * c + 1)
        plsc.store_scatter(jw_buf, [i0], j0, mask=m0)
        plsc.store_scatter(jw_buf, [i1], j1, mask=m1)
        return 0

    lax.fori_loop(0, (n + 2 * L - 1) // (2 * L), scan_pair, 0)

    def comp_body(i, off):
        v = jw_buf[pl.ds(i * L, L)]
        m = v >= 0
        mi = m.astype(jnp.int32)
        pos = off + plsc.cumsum(mi) - 1
        t = lo + i * L + lane
        plsc.store_scatter(tl128, [pos >> 7, pos & 127], t, mask=m)
        plsc.store_scatter(jl128, [pos >> 7, pos & 127], v, mask=m)
        plsc.store_scatter(tl16, [pos >> 4, pos & 15], t, mask=m)
        plsc.store_scatter(jl16, [pos >> 4, pos & 15], v, mask=m)
        return off + jnp.sum(mi)

    cnt = lax.fori_loop(0, tpb // L, comp_body, jnp.int32(0))

    cntbuf[...] = _splat(cnt)
    pltpu.sync_copy(tl128, tl128_hbm.at[wid])
    pltpu.sync_copy(jl128, jl128_hbm.at[wid])
    pltpu.sync_copy(tl16, tl16_hbm.at[wid])
    pltpu.sync_copy(jl16, jl16_hbm.at[wid])
    pltpu.sync_copy(cntbuf, cnt_hbm.at[wid])


def _invert(idx, n, tp, zbase):
    tpb = tp // NW
    ch128 = (tpb + 127) // 128
    ch16 = tpb // L
    body = functools.partial(_invert_body, n=n, tpb=tpb, ch128=ch128,
                             ch16=ch16, zbase=zbase)
    i32 = jnp.int32
    return pl.kernel(
        body,
        out_type=(
            jax.ShapeDtypeStruct((NW, ch128, 128), i32),
            jax.ShapeDtypeStruct((NW, ch128, 128), i32),
            jax.ShapeDtypeStruct((NW, ch16, L), i32),
            jax.ShapeDtypeStruct((NW, ch16, L), i32),
            jax.ShapeDtypeStruct((NW, L), i32),
        ),
        mesh=plsc.VectorSubcoreMesh(**_MESH),
        scratch_types=[
            pltpu.VMEM((25024,), i32),
            pltpu.VMEM((tpb,), i32),
            pltpu.VMEM((ch128, 128), i32),
            pltpu.VMEM((ch128, 128), i32),
            pltpu.VMEM((ch16, L), i32),
            pltpu.VMEM((ch16, L), i32),
            pltpu.VMEM((L,), i32),
        ],
        compiler_params=pltpu.CompilerParams(needs_layout_passes=False, use_tc_tiling_on_sc=True),
    )(idx)


def _rmw_body(tl128_hbm, jl128_hbm, tl16_hbm, jl16_hbm, cnt_hbm, *gh_args,
              ch128, ch16, npairs):
    gh_pairs = [(gh_args[2 * i], gh_args[2 * i + 1]) for i in range(npairs)]
    (tl128, jl128, tl16, jl16, cntbuf, bufG, bufH, bufG2, bufH2,
     semG, semH, semS, semG2, semH2, semS2) = gh_args[2 * npairs:]
    wid = _wid()
    lane = _lane()
    pltpu.sync_copy(tl128_hbm.at[wid], tl128)
    pltpu.sync_copy(jl128_hbm.at[wid], jl128)
    pltpu.sync_copy(tl16_hbm.at[wid], tl16)
    pltpu.sync_copy(jl16_hbm.at[wid], jl16)
    pltpu.sync_copy(cnt_hbm.at[wid], cntbuf)
    cnt = jnp.max(cntbuf[...])
    n128 = cnt >> 7
    n16 = (cnt & 127) >> 4
    rem = cnt & 15

    def add_rows(bh, bg, nrows):
        def row_body(r, _):
            for c in range(8):
                s = pl.ds(c * L, L)
                bh[r, s] = bh[r, s] + bg[r, s]
            return 0
        lax.fori_loop(0, nrows, row_body, 0)

    for g_hbm, h_ref in gh_pairs:
        _rmw_one(g_hbm, h_ref, tl128, jl128, tl16, jl16, lane, n128, n16, rem,
                 bufG, bufH, bufG2, bufH2, semG, semH, semS, semG2, semH2,
                 semS2, add_rows)


def _rmw_one(g_hbm, h_ref, tl128, jl128, tl16, jl16, lane, n128, n16, rem,
             bufG, bufH, bufG2, bufH2, semG, semH, semS, semG2, semH2, semS2,
             add_rows):
    def big_pair(p, _):
        k0 = 2 * p
        k1 = 2 * p + 1
        gh0 = pltpu.async_copy(g_hbm.at[jl128.at[k0]], bufG, semG)
        hh0 = pltpu.async_copy(h_ref.at[tl128.at[k0]], bufH, semH)
        gh1 = pltpu.async_copy(g_hbm.at[jl128.at[k1]], bufG2, semG2)
        hh1 = pltpu.async_copy(h_ref.at[tl128.at[k1]], bufH2, semH2)
        gh0.wait()
        hh0.wait()
        add_rows(bufH, bufG, 128)
        sc0 = pltpu.async_copy(bufH, h_ref.at[tl128.at[k0]], semS)
        gh1.wait()
        hh1.wait()
        add_rows(bufH2, bufG2, 128)
        sc1 = pltpu.async_copy(bufH2, h_ref.at[tl128.at[k1]], semS2)
        sc0.wait()
        sc1.wait()
        return 0

    lax.fori_loop(0, n128 >> 1, big_pair, 0)

    def big_last(k, _):
        gh = pltpu.async_copy(g_hbm.at[jl128.at[k]], bufG, semG)
        hh = pltpu.async_copy(h_ref.at[tl128.at[k]], bufH, semH)
        gh.wait()
        hh.wait()
        add_rows(bufH, bufG, 128)
        pltpu.async_copy(bufH, h_ref.at[tl128.at[k]], semS).wait()
        return 0

    lax.fori_loop(n128 & ~1, n128, big_last, 0)

    def mid_chunk(i, _):
        k = n128 * 8 + i
        gh = pltpu.async_copy(g_hbm.at[jl16.at[k]], bufG.at[pl.ds(0, L)],
                              semG)
        hh = pltpu.async_copy(h_ref.at[tl16.at[k]], bufH.at[pl.ds(0, L)],
                              semH)
        gh.wait()
        hh.wait()
        add_rows(bufH, bufG, L)
        pltpu.async_copy(bufH.at[pl.ds(0, L)], h_ref.at[tl16.at[k]],
                         semS).wait()
        return 0

    lax.fori_loop(0, n16, mid_chunk, 0)

    ktail = n128 * 8 + n16
    tvec = plsc.load_gather(tl16, [_splat(ktail), lane])
    jvec = plsc.load_gather(jl16, [_splat(ktail), lane])

    def scalar_at(vec, e):
        return jnp.max(jnp.where(lane == e, vec, -1))

    for e in range(15):
        @pl.when(e < rem)
        def _start():
            j_e = scalar_at(jvec, e)
            t_e = scalar_at(tvec, e)
            pltpu.make_async_copy(g_hbm.at[pl.ds(j_e, 1)],
                                  bufG.at[pl.ds(e, 1)], semG).start()
            pltpu.make_async_copy(h_ref.at[pl.ds(t_e, 1)],
                                  bufH.at[pl.ds(e, 1)], semH).start()

    for e in range(15):
        @pl.when(e < rem)
        def _apply():
            pltpu.make_async_copy(g_hbm.at[pl.ds(0, 1)],
                                  bufG.at[pl.ds(e, 1)], semG).wait()
            pltpu.make_async_copy(g_hbm.at[pl.ds(0, 1)],
                                  bufH.at[pl.ds(e, 1)], semH).wait()
            for c in range(8):
                s = pl.ds(c * L, L)
                bufH[e, s] = bufH[e, s] + bufG[e, s]
            t_e = scalar_at(tvec, e)
            pltpu.make_async_copy(bufH.at[pl.ds(e, 1)],
                                  h_ref.at[pl.ds(t_e, 1)], semS).start()

    for e in range(15):
        @pl.when(e < rem)
        def _drain():
            pltpu.make_async_copy(bufH.at[pl.ds(e, 1)],
                                  h_ref.at[pl.ds(0, 1)], semS).wait()


def _rmw(lists, gh_pairs, tp):
    tpb = tp // NW
    ch128 = (tpb + 127) // 128
    ch16 = tpb // L
    body = functools.partial(_rmw_body, ch128=ch128, ch16=ch16,
                             npairs=len(gh_pairs))
    i32 = jnp.int32
    flat = [x for gh in gh_pairs for x in gh]
    pl.kernel(
        body,
        out_type=(),
        mesh=plsc.VectorSubcoreMesh(**_MESH),
        scratch_types=[
            pltpu.VMEM((ch128, 128), i32),
            pltpu.VMEM((ch128, 128), i32),
            pltpu.VMEM((ch16, L), i32),
            pltpu.VMEM((ch16, L), i32),
            pltpu.VMEM((L,), i32),
            pltpu.VMEM((128, D), jnp.float32),
            pltpu.VMEM((128, D), jnp.float32),
            pltpu.VMEM((128, D), jnp.float32),
            pltpu.VMEM((128, D), jnp.float32),
            pltpu.SemaphoreType.DMA,
            pltpu.SemaphoreType.DMA,
            pltpu.SemaphoreType.DMA,
            pltpu.SemaphoreType.DMA,
            pltpu.SemaphoreType.DMA,
            pltpu.SemaphoreType.DMA,
        ],
        compiler_params=pltpu.CompilerParams(needs_layout_passes=False, use_tc_tiling_on_sc=True),
    )(*lists, *flat)


def _prep_body(w0, b0, bup0, w0b, w1, b1, bup1, w1b, w1t, w0t,
               w0e, b0e, w1e, b1e, wc):
    f32 = jnp.float32
    w0e[...] = w0[...] + jnp.dot(w0[...], w0b[...], preferred_element_type=f32)
    b0e[...] = b0[...] + jnp.dot(b0[...], w0b[...], preferred_element_type=f32) + bup0[...]
    w1e[...] = w1[...] + jnp.dot(w1[...], w1b[...], preferred_element_type=f32)
    b1e[...] = b1[...] + jnp.dot(b1[...], w1b[...], preferred_element_type=f32) + bup1[...]
    wc[...] = jnp.dot(w1t[...], w0t[...], preferred_element_type=f32)


def _prep(W0, b0r, bup0r, Wup0b, W1, b1r, bup1r, Wup1b, Wup1t, Wup0t):
    f32 = jnp.float32
    return pl.pallas_call(
        _prep_body,
        out_shape=(
            jax.ShapeDtypeStruct((D, D), f32),
            jax.ShapeDtypeStruct((1, D), f32),
            jax.ShapeDtypeStruct((D, D), f32),
            jax.ShapeDtypeStruct((1, D), f32),
            jax.ShapeDtypeStruct((D, D), f32),
        ),
    )(W0, b0r, bup0r, Wup0b, W1, b1r, bup1r, Wup1b, Wup1t, Wup0t)


def _tc2_body(hn2_ref, w2_ref, b2_ref, wt_ref, wc_ref, h2_ref, g2_ref,
              gc2_ref):
    g = pl.program_id(0)
    rows = g * BLK + lax.broadcasted_iota(jnp.int32, (BLK, 1), 0)
    h = jnp.dot(hn2_ref[...], w2_ref[...],
                preferred_element_type=jnp.float32) + b2_ref[...]
    h2_ref[...] = h
    gv = jnp.dot(h, wt_ref[...], preferred_element_type=jnp.float32)
    g2_ref[...] = jnp.where(rows < N2, gv, 0.0)
    gcv = jnp.dot(h, wc_ref[...], preferred_element_type=jnp.float32)
    gc2_ref[...] = jnp.where(rows < N2, gcv, 0.0)


def _tc2(hn2, W2, b2r, Wup1t, Wc):
    nb = N2P // BLK
    return pl.pallas_call(
        _tc2_body,
        grid=(nb,),
        in_specs=[
            pl.BlockSpec((BLK, D), lambda i: (i, 0)),
            pl.BlockSpec((D, D), lambda i: (0, 0)),
            pl.BlockSpec((1, D), lambda i: (0, 0)),
            pl.BlockSpec((D, D), lambda i: (0, 0)),
            pl.BlockSpec((D, D), lambda i: (0, 0)),
        ],
        out_specs=(
            pl.BlockSpec((BLK, D), lambda i: (i, 0)),
            pl.BlockSpec((BLK, D), lambda i: (i, 0)),
            pl.BlockSpec((BLK, D), lambda i: (i, 0)),
        ),
        out_shape=(
            jax.ShapeDtypeStruct((N2, D), jnp.float32),
            jax.ShapeDtypeStruct((N2P, D), jnp.float32),
            jax.ShapeDtypeStruct((N2P, D), jnp.float32),
        ),
    )(hn2, W2, b2r, Wup1t, Wc)


def _dense_body(hn_ref, w_ref, b_ref, out_ref):
    out_ref[...] = jnp.dot(hn_ref[...], w_ref[...],
                           preferred_element_type=jnp.float32) + b_ref[...]


def _dense(hn, We, be, nrows, blk=BLK):
    return pl.pallas_call(
        _dense_body,
        grid=(nrows // blk,),
        in_specs=[
            pl.BlockSpec((blk, D), lambda i: (i, 0)),
            pl.BlockSpec((D, D), lambda i: (0, 0)),
            pl.BlockSpec((1, D), lambda i: (0, 0)),
        ],
        out_specs=pl.BlockSpec((blk, D), lambda i: (i, 0)),
        out_shape=jax.ShapeDtypeStruct((nrows, D), jnp.float32),
    )(hn, We, be)


def _dense1g_body(hn_ref, w_ref, b_ref, wt_ref, h1_ref, g1_ref):
    nb = N1P // BLK
    r = (nb - 1) - pl.program_id(0)
    rows = r * BLK + lax.broadcasted_iota(jnp.int32, (BLK, 1), 0)
    h = jnp.dot(hn_ref[...], w_ref[...],
                preferred_element_type=jnp.float32) + b_ref[...]
    h1_ref[...] = h
    gv = jnp.dot(h, wt_ref[...], preferred_element_type=jnp.float32)
    g1_ref[...] = jnp.where(rows < N1, gv, 0.0)


def _dense1g(hn1, We, be, Wup0t):
    nb = N1P // BLK
    rev = nb - 1
    return pl.pallas_call(
        _dense1g_body,
        grid=(nb,),
        in_specs=[
            pl.BlockSpec((BLK, D), lambda i: (jnp.minimum(rev - i, rev - 1), 0)),
            pl.BlockSpec((D, D), lambda i: (0, 0)),
            pl.BlockSpec((1, D), lambda i: (0, 0)),
            pl.BlockSpec((D, D), lambda i: (0, 0)),
        ],
        out_specs=(
            pl.BlockSpec((BLK, D), lambda i: (jnp.minimum(rev - i, rev - 1), 0)),
            pl.BlockSpec((BLK, D), lambda i: (rev - i, 0)),
        ),
        out_shape=(
            jax.ShapeDtypeStruct((N1, D), jnp.float32),
            jax.ShapeDtypeStruct((N1P, D), jnp.float32),
        ),
    )(hn1, We, be, Wup0t)


def kernel(hn0, hn1, hn2, idx1, idx2, W0, b0, W1, b1, W2, b2, Wup0, bup0,
           Wup1, bup1):
    b0r, b1r, b2r = b0.reshape(1, D), b1.reshape(1, D), b2.reshape(1, D)
    bup0r, bup1r = bup0.reshape(1, D), bup1.reshape(1, D)
    Wup0t, Wup0b = Wup0[:D], Wup0[D:]
    Wup1t, Wup1b = Wup1[:D], Wup1[D:]

    W0e, b0e, W1e, b1e, Wc = _prep(W0, b0r, bup0r, Wup0b, W1, b1r, bup1r,
                                   Wup1b, Wup1t, Wup0t)

    lists2 = _invert(idx2, N2, T1P, zbase=N2)
    lists1 = _invert(idx1, N1, T0P, zbase=N1)

    h2, g2pad, gc2pad = _tc2(hn2, W2, b2r, Wup1t, Wc)
    h1d, g1d = _dense1g(hn1, W1e, b1e, Wup0t)
    h0d = _dense(hn0, W0e, b0e, N0, blk=2000)

    h1_ref = jax.new_ref(h1d)
    g1_ref = jax.new_ref(g1d)
    _rmw(lists2, [(g2pad, h1_ref), (gc2pad, g1_ref)], T1P)
    h1 = jax.freeze(h1_ref)
    g1pad = jax.freeze(g1_ref)

    h0_ref = jax.new_ref(h0d)
    _rmw(lists1, [(g1pad, h0_ref)], T0P)
    h0 = jax.freeze(h0_ref)

    return (h0, h1, h2)


# speedup vs baseline: 1.3358x; 1.0376x over previous
import functools

import jax
import jax.numpy as jnp
from jax import lax
from jax.experimental import pallas as pl
from jax.experimental.pallas import tpu as pltpu
from jax.experimental.pallas import tpu_sc as plsc

N0, N1, N2, D = 100000, 25000, 6250, 128
L = 16
NW = 32
BLK = 1000

N2P, N1P = 7000, 26000
T1P = 25088
T0P = 100352
SENT = 1 << 26

_MESH = dict(core_axis_name="c", subcore_axis_name="s")


def _wid():
    return lax.axis_index("s") * 2 + lax.axis_index("c")


def _lane():
    return lax.broadcasted_iota(jnp.int32, (L,), 0)


def _splat(x):
    return jnp.broadcast_to(x, (L,)).astype(jnp.int32)


def _invert_body(idx_hbm, tl128_hbm, jl128_hbm, tl16_hbm, jl16_hbm, cnt_hbm,
                 idx_buf, jw_buf, tl128, jl128, tl16, jl16, cntbuf,
                 *, n, tpb, ch128, ch16, zbase):
    wid = _wid()
    lo = wid * tpb
    lane = _lane()
    zvec = zbase + wid * L + lane

    pltpu.sync_copy(idx_hbm, idx_buf.at[pl.ds(0, n)])
    nfloor = n - n % L
    tail = idx_buf[pl.ds(nfloor, L)]
    idx_buf[pl.ds(nfloor, L)] = jnp.where(lane < n - nfloor, tail, SENT)
    idx_buf[pl.ds(nfloor + L, L)] = jnp.full((L,), SENT, jnp.int32)

    minus1 = jnp.full((L,), -1, jnp.int32)

    def init_jw(i, _):
        jw_buf[pl.ds(i * L, L)] = minus1
        return 0

    lax.fori_loop(0, tpb // L, init_jw, 0)

    def init_lists(i, _):
        for c in range(8):
            tl128[i, pl.ds(c * L, L)] = jnp.zeros((L,), jnp.int32)
            jl128[i, pl.ds(c * L, L)] = zvec
        return 0

    lax.fori_loop(0, ch128, init_lists, 0)

    def init_lists16(i, _):
        tl16[i, :] = jnp.zeros((L,), jnp.int32)
        jl16[i, :] = zvec
        return 0

    lax.fori_loop(0, ch16, init_lists16, 0)

    perm = jnp.minimum(lane + 1, L - 1)

    def scan_one(cc):
        v = idx_buf[pl.ds(cc * L, L)]
        key = v * L + lane
        skey = plsc.sort_key_val(key, key)[0]
        tgt = skey >> 4
        nxt = skey.at[perm].get(mode="promise_in_bounds")
        keep = (tgt != (nxt >> 4)) | (lane == L - 1)
        m = keep & (tgt >= lo) & (tgt < lo + tpb)
        jval = cc * L + (skey & (L - 1))
        return tgt - lo, jval, m

    def scan_pair(c, _):
        i0, j0, m0 = scan_one(2 * c)
        i1, j1, m1 = scan_one(2 * c + 1)
        plsc.store_scatter(jw_buf, [i0], j0, mask=m0)
        plsc.store_scatter(jw_buf, [i1], j1, mask=m1)
        return 0

    lax.fori_loop(0, (n + 2 * L - 1) // (2 * L), scan_pair, 0)

    def comp_body(i, off):
        v = jw_buf[pl.ds(i * L, L)]
        m = v >= 0
        mi = m.astype(jnp.int32)
        pos = off + plsc.cumsum(mi) - 1
        t = lo + i * L + lane
        plsc.store_scatter(tl128, [pos >> 7, pos & 127], t, mask=m)
        plsc.store_scatter(jl128, [pos >> 7, pos & 127], v, mask=m)
        plsc.store_scatter(tl16, [pos >> 4, pos & 15], t, mask=m)
        plsc.store_scatter(jl16, [pos >> 4, pos & 15], v, mask=m)
        return off + jnp.sum(mi)

    cnt = lax.fori_loop(0, tpb // L, comp_body, jnp.int32(0))

    cntbuf[...] = _splat(cnt)
    pltpu.sync_copy(tl128, tl128_hbm.at[wid])
    pltpu.sync_copy(jl128, jl128_hbm.at[wid])
    pltpu.sync_copy(tl16, tl16_hbm.at[wid])
    pltpu.sync_copy(jl16, jl16_hbm.at[wid])
    pltpu.sync_copy(cntbuf, cnt_hbm.at[wid])


def _invert(idx, n, tp, zbase):
    tpb = tp // NW
    ch128 = (tpb + 127) // 128
    ch16 = tpb // L
    body = functools.partial(_invert_body, n=n, tpb=tpb, ch128=ch128,
                             ch16=ch16, zbase=zbase)
    i32 = jnp.int32
    return pl.kernel(
        body,
        out_type=(
            jax.ShapeDtypeStruct((NW, ch128, 128), i32),
            jax.ShapeDtypeStruct((NW, ch128, 128), i32),
            jax.ShapeDtypeStruct((NW, ch16, L), i32),
            jax.ShapeDtypeStruct((NW, ch16, L), i32),
            jax.ShapeDtypeStruct((NW, L), i32),
        ),
        mesh=plsc.VectorSubcoreMesh(**_MESH),
        scratch_types=[
            pltpu.VMEM((25024,), i32),
            pltpu.VMEM((tpb,), i32),
            pltpu.VMEM((ch128, 128), i32),
            pltpu.VMEM((ch128, 128), i32),
            pltpu.VMEM((ch16, L), i32),
            pltpu.VMEM((ch16, L), i32),
            pltpu.VMEM((L,), i32),
        ],
        compiler_params=pltpu.CompilerParams(needs_layout_passes=False, use_tc_tiling_on_sc=True),
    )(idx)


def _rmw_body(tl128_hbm, jl128_hbm, tl16_hbm, jl16_hbm, cnt_hbm, *gh_args,
              ch128, ch16, npairs):
    gh_pairs = [(gh_args[2 * i], gh_args[2 * i + 1]) for i in range(npairs)]
    (tl128, jl128, tl16, jl16, cntbuf, bufG, bufH, bufG2, bufH2,
     semG, semH, semS, semG2, semH2, semS2) = gh_args[2 * npairs:]
    wid = _wid()
    lane = _lane()
    pltpu.sync_copy(tl128_hbm.at[wid], tl128)
    pltpu.sync_copy(jl128_hbm.at[wid], jl128)
    pltpu.sync_copy(tl16_hbm.at[wid], tl16)
    pltpu.sync_copy(jl16_hbm.at[wid], jl16)
    pltpu.sync_copy(cnt_hbm.at[wid], cntbuf)
    cnt = jnp.max(cntbuf[...])
    n128 = cnt >> 7
    n16 = (cnt & 127) >> 4
    rem = cnt & 15

    def add_rows(bh, bg, nrows):
        def row_body(r, _):
            for c in range(8):
                s = pl.ds(c * L, L)
                bh[r, s] = bh[r, s] + bg[r, s]
            return 0
        lax.fori_loop(0, nrows, row_body, 0)

    for g_hbm, h_ref in gh_pairs:
        _rmw_one(g_hbm, h_ref, tl128, jl128, tl16, jl16, lane, n128, n16, rem,
                 bufG, bufH, bufG2, bufH2, semG, semH, semS, semG2, semH2,
                 semS2, add_rows)


def _rmw_one(g_hbm, h_ref, tl128, jl128, tl16, jl16, lane, n128, n16, rem,
             bufG, bufH, bufG2, bufH2, semG, semH, semS, semG2, semH2, semS2,
             add_rows):
    def big_pair(p, _):
        k0 = 2 * p
        k1 = 2 * p + 1
        gh0 = pltpu.async_copy(g_hbm.at[jl128.at[k0]], bufG, semG)
        hh0 = pltpu.async_copy(h_ref.at[tl128.at[k0]], bufH, semH)
        gh1 = pltpu.async_copy(g_hbm.at[jl128.at[k1]], bufG2, semG2)
        hh1 = pltpu.async_copy(h_ref.at[tl128.at[k1]], bufH2, semH2)
        gh0.wait()
        hh0.wait()
        add_rows(bufH, bufG, 128)
        sc0 = pltpu.async_copy(bufH, h_ref.at[tl128.at[k0]], semS)
        gh1.wait()
        hh1.wait()
        add_rows(bufH2, bufG2, 128)
        sc1 = pltpu.async_copy(bufH2, h_ref.at[tl128.at[k1]], semS2)
        sc0.wait()
        sc1.wait()
        return 0

    lax.fori_loop(0, n128 >> 1, big_pair, 0)

    def big_last(k, _):
        gh = pltpu.async_copy(g_hbm.at[jl128.at[k]], bufG, semG)
        hh = pltpu.async_copy(h_ref.at[tl128.at[k]], bufH, semH)
        gh.wait()
        hh.wait()
        add_rows(bufH, bufG, 128)
        pltpu.async_copy(bufH, h_ref.at[tl128.at[k]], semS).wait()
        return 0

    lax.fori_loop(n128 & ~1, n128, big_last, 0)

    def mid_chunk(i, _):
        k = n128 * 8 + i
        gh = pltpu.async_copy(g_hbm.at[jl16.at[k]], bufG.at[pl.ds(0, L)],
                              semG)
        hh = pltpu.async_copy(h_ref.at[tl16.at[k]], bufH.at[pl.ds(0, L)],
                              semH)
        gh.wait()
        hh.wait()
        add_rows(bufH, bufG, L)
        pltpu.async_copy(bufH.at[pl.ds(0, L)], h_ref.at[tl16.at[k]],
                         semS).wait()
        return 0

    lax.fori_loop(0, n16, mid_chunk, 0)

    ktail = n128 * 8 + n16
    tvec = plsc.load_gather(tl16, [_splat(ktail), lane])
    jvec = plsc.load_gather(jl16, [_splat(ktail), lane])

    def scalar_at(vec, e):
        return jnp.max(jnp.where(lane == e, vec, -1))

    for e in range(15):
        @pl.when(e < rem)
        def _start():
            j_e = scalar_at(jvec, e)
            t_e = scalar_at(tvec, e)
            pltpu.make_async_copy(g_hbm.at[pl.ds(j_e, 1)],
                                  bufG.at[pl.ds(e, 1)], semG).start()
            pltpu.make_async_copy(h_ref.at[pl.ds(t_e, 1)],
                                  bufH.at[pl.ds(e, 1)], semH).start()

    for e in range(15):
        @pl.when(e < rem)
        def _apply():
            pltpu.make_async_copy(g_hbm.at[pl.ds(0, 1)],
                                  bufG.at[pl.ds(e, 1)], semG).wait()
            pltpu.make_async_copy(g_hbm.at[pl.ds(0, 1)],
                                  bufH.at[pl.ds(e, 1)], semH).wait()
            for c in range(8):
                s = pl.ds(c * L, L)
                bufH[e, s] = bufH[e, s] + bufG[e, s]
            t_e = scalar_at(tvec, e)
            pltpu.make_async_copy(bufH.at[pl.ds(e, 1)],
                                  h_ref.at[pl.ds(t_e, 1)], semS).start()

    for e in range(15):
        @pl.when(e < rem)
        def _drain():
            pltpu.make_async_copy(bufH.at[pl.ds(e, 1)],
                                  h_ref.at[pl.ds(0, 1)], semS).wait()


def _rmw(lists, gh_pairs, tp):
    tpb = tp // NW
    ch128 = (tpb + 127) // 128
    ch16 = tpb // L
    body = functools.partial(_rmw_body, ch128=ch128, ch16=ch16,
                             npairs=len(gh_pairs))
    i32 = jnp.int32
    flat = [x for gh in gh_pairs for x in gh]
    pl.kernel(
        body,
        out_type=(),
        mesh=plsc.VectorSubcoreMesh(**_MESH),
        scratch_types=[
            pltpu.VMEM((ch128, 128), i32),
            pltpu.VMEM((ch128, 128), i32),
            pltpu.VMEM((ch16, L), i32),
            pltpu.VMEM((ch16, L), i32),
            pltpu.VMEM((L,), i32),
            pltpu.VMEM((128, D), jnp.float32),
            pltpu.VMEM((128, D), jnp.float32),
            pltpu.VMEM((128, D), jnp.float32),
            pltpu.VMEM((128, D), jnp.float32),
            pltpu.SemaphoreType.DMA,
            pltpu.SemaphoreType.DMA,
            pltpu.SemaphoreType.DMA,
            pltpu.SemaphoreType.DMA,
            pltpu.SemaphoreType.DMA,
            pltpu.SemaphoreType.DMA,
        ],
        compiler_params=pltpu.CompilerParams(needs_layout_passes=False, use_tc_tiling_on_sc=True),
    )(*lists, *flat)


def _prep_body(w0, b0, bup0, w0b, w1, b1, bup1, w1b, w1t, w0t,
               w0e, b0e, w1e, b1e, wc):
    f32 = jnp.float32
    w0e[...] = w0[...] + jnp.dot(w0[...], w0b[...], preferred_element_type=f32)
    b0e[...] = b0[...] + jnp.dot(b0[...], w0b[...], preferred_element_type=f32) + bup0[...]
    w1e[...] = w1[...] + jnp.dot(w1[...], w1b[...], preferred_element_type=f32)
    b1e[...] = b1[...] + jnp.dot(b1[...], w1b[...], preferred_element_type=f32) + bup1[...]
    wc[...] = jnp.dot(w1t[...], w0t[...], preferred_element_type=f32)


def _prep(W0, b0r, bup0r, Wup0b, W1, b1r, bup1r, Wup1b, Wup1t, Wup0t):
    f32 = jnp.float32
    return pl.pallas_call(
        _prep_body,
        out_shape=(
            jax.ShapeDtypeStruct((D, D), f32),
            jax.ShapeDtypeStruct((1, D), f32),
            jax.ShapeDtypeStruct((D, D), f32),
            jax.ShapeDtypeStruct((1, D), f32),
            jax.ShapeDtypeStruct((D, D), f32),
        ),
    )(W0, b0r, bup0r, Wup0b, W1, b1r, bup1r, Wup1b, Wup1t, Wup0t)


def _tc2_body(hn2_ref, w2_ref, b2_ref, wt_ref, wc_ref, h2_ref, g2_ref,
              gc2_ref):
    g = pl.program_id(0)
    rows = g * BLK + lax.broadcasted_iota(jnp.int32, (BLK, 1), 0)
    h = jnp.dot(hn2_ref[...], w2_ref[...],
                preferred_element_type=jnp.float32) + b2_ref[...]
    h2_ref[...] = h
    gv = jnp.dot(h, wt_ref[...], preferred_element_type=jnp.float32)
    g2_ref[...] = jnp.where(rows < N2, gv, 0.0)
    gcv = jnp.dot(h, wc_ref[...], preferred_element_type=jnp.float32)
    gc2_ref[...] = jnp.where(rows < N2, gcv, 0.0)


def _tc2(hn2, W2, b2r, Wup1t, Wc):
    nb = N2P // BLK
    return pl.pallas_call(
        _tc2_body,
        grid=(nb,),
        in_specs=[
            pl.BlockSpec((BLK, D), lambda i: (i, 0)),
            pl.BlockSpec((D, D), lambda i: (0, 0)),
            pl.BlockSpec((1, D), lambda i: (0, 0)),
            pl.BlockSpec((D, D), lambda i: (0, 0)),
            pl.BlockSpec((D, D), lambda i: (0, 0)),
        ],
        out_specs=(
            pl.BlockSpec((BLK, D), lambda i: (i, 0)),
            pl.BlockSpec((BLK, D), lambda i: (i, 0)),
            pl.BlockSpec((BLK, D), lambda i: (i, 0)),
        ),
        out_shape=(
            jax.ShapeDtypeStruct((N2, D), jnp.float32),
            jax.ShapeDtypeStruct((N2P, D), jnp.float32),
            jax.ShapeDtypeStruct((N2P, D), jnp.float32),
        ),
    )(hn2, W2, b2r, Wup1t, Wc)


def _dense_body(hn_ref, w_ref, b_ref, out_ref):
    out_ref[...] = jnp.dot(hn_ref[...], w_ref[...],
                           preferred_element_type=jnp.float32) + b_ref[...]


def _dense(hn, We, be, nrows, blk=BLK):
    return pl.pallas_call(
        _dense_body,
        grid=(nrows // blk,),
        in_specs=[
            pl.BlockSpec((blk, D), lambda i: (i, 0)),
            pl.BlockSpec((D, D), lambda i: (0, 0)),
            pl.BlockSpec((1, D), lambda i: (0, 0)),
        ],
        out_specs=pl.BlockSpec((blk, D), lambda i: (i, 0)),
        out_shape=jax.ShapeDtypeStruct((nrows, D), jnp.float32),
    )(hn, We, be)


def _dense1g_body(hn_ref, w_ref, b_ref, wt_ref, h1_ref, g1_ref):
    nb = N1P // BLK
    r = (nb - 1) - pl.program_id(0)
    rows = r * BLK + lax.broadcasted_iota(jnp.int32, (BLK, 1), 0)
    h = jnp.dot(hn_ref[...], w_ref[...],
                preferred_element_type=jnp.float32) + b_ref[...]
    h1_ref[...] = h
    gv = jnp.dot(h, wt_ref[...], preferred_element_type=jnp.float32)
    g1_ref[...] = jnp.where(rows < N1, gv, 0.0)


def _dense1g(hn1, We, be, Wup0t):
    nb = N1P // BLK
    rev = nb - 1
    return pl.pallas_call(
        _dense1g_body,
        grid=(nb,),
        in_specs=[
            pl.BlockSpec((BLK, D), lambda i: (jnp.minimum(rev - i, rev - 1), 0)),
            pl.BlockSpec((D, D), lambda i: (0, 0)),
            pl.BlockSpec((1, D), lambda i: (0, 0)),
            pl.BlockSpec((D, D), lambda i: (0, 0)),
        ],
        out_specs=(
            pl.BlockSpec((BLK, D), lambda i: (jnp.minimum(rev - i, rev - 1), 0)),
            pl.BlockSpec((BLK, D), lambda i: (rev - i, 0)),
        ),
        out_shape=(
            jax.ShapeDtypeStruct((N1, D), jnp.float32),
            jax.ShapeDtypeStruct((N1P, D), jnp.float32),
        ),
    )(hn1, We, be, Wup0t)


def kernel(hn0, hn1, hn2, idx1, idx2, W0, b0, W1, b1, W2, b2, Wup0, bup0,
           Wup1, bup1):
    b0r, b1r, b2r = b0.reshape(1, D), b1.reshape(1, D), b2.reshape(1, D)
    bup0r, bup1r = bup0.reshape(1, D), bup1.reshape(1, D)
    Wup0t, Wup0b = Wup0[:D], Wup0[D:]
    Wup1t, Wup1b = Wup1[:D], Wup1[D:]

    W0e, b0e, W1e, b1e, Wc = _prep(W0, b0r, bup0r, Wup0b, W1, b1r, bup1r,
                                   Wup1b, Wup1t, Wup0t)

    lists2 = _invert(idx2, N2, T1P, zbase=N2)
    lists1 = _invert(idx1, N1, T0P, zbase=N1)

    h2, g2pad, gc2pad = _tc2(hn2, W2, b2r, Wup1t, Wc)
    h1d, g1d = _dense1g(hn1, W1e, b1e, Wup0t)
    h0d = _dense(hn0, W0e, b0e, N0, blk=5000)

    h1_ref = jax.new_ref(h1d)
    g1_ref = jax.new_ref(g1d)
    _rmw(lists2, [(g2pad, h1_ref), (gc2pad, g1_ref)], T1P)
    h1 = jax.freeze(h1_ref)
    g1pad = jax.freeze(g1_ref)

    h0_ref = jax.new_ref(h0d)
    _rmw(lists1, [(g1pad, h0_ref)], T0P)
    h0 = jax.freeze(h0_ref)

    return (h0, h1, h2)


# speedup vs baseline: 1.3364x; 1.0004x over previous
import functools

import jax
import jax.numpy as jnp
from jax import lax
from jax.experimental import pallas as pl
from jax.experimental.pallas import tpu as pltpu
from jax.experimental.pallas import tpu_sc as plsc

N0, N1, N2, D = 100000, 25000, 6250, 128
L = 16
NW = 32
BLK = 1000

N2P, N1P = 7000, 26000
T1P = 25088
T0P = 100352
SENT = 1 << 26

_MESH = dict(core_axis_name="c", subcore_axis_name="s")


def _wid():
    return lax.axis_index("s") * 2 + lax.axis_index("c")


def _lane():
    return lax.broadcasted_iota(jnp.int32, (L,), 0)


def _splat(x):
    return jnp.broadcast_to(x, (L,)).astype(jnp.int32)


def _invert_body(idx_hbm, tl128_hbm, jl128_hbm, tl16_hbm, jl16_hbm, cnt_hbm,
                 idx_buf, jw_buf, tl128, jl128, tl16, jl16, cntbuf,
                 *, n, tpb, ch128, ch16, zbase):
    wid = _wid()
    lo = wid * tpb
    lane = _lane()
    zvec = zbase + wid * L + lane

    pltpu.sync_copy(idx_hbm, idx_buf.at[pl.ds(0, n)])
    nfloor = n - n % L
    tail = idx_buf[pl.ds(nfloor, L)]
    idx_buf[pl.ds(nfloor, L)] = jnp.where(lane < n - nfloor, tail, SENT)
    idx_buf[pl.ds(nfloor + L, L)] = jnp.full((L,), SENT, jnp.int32)

    minus1 = jnp.full((L,), -1, jnp.int32)

    def init_jw(i, _):
        jw_buf[pl.ds(i * L, L)] = minus1
        return 0

    lax.fori_loop(0, tpb // L, init_jw, 0)

    def init_lists(i, _):
        for c in range(8):
            tl128[i, pl.ds(c * L, L)] = jnp.zeros((L,), jnp.int32)
            jl128[i, pl.ds(c * L, L)] = zvec
        return 0

    lax.fori_loop(0, ch128, init_lists, 0)

    def init_lists16(i, _):
        tl16[i, :] = jnp.zeros((L,), jnp.int32)
        jl16[i, :] = zvec
        return 0

    lax.fori_loop(0, ch16, init_lists16, 0)

    perm = jnp.minimum(lane + 1, L - 1)

    def scan_one(cc):
        v = idx_buf[pl.ds(cc * L, L)]
        key = v * L + lane
        skey = plsc.sort_key_val(key, key)[0]
        tgt = skey >> 4
        nxt = skey.at[perm].get(mode="promise_in_bounds")
        keep = (tgt != (nxt >> 4)) | (lane == L - 1)
        m = keep & (tgt >= lo) & (tgt < lo + tpb)
        jval = cc * L + (skey & (L - 1))
        return tgt - lo, jval, m

    def scan_pair(c, _):
        i0, j0, m0 = scan_one(2 * c)
        i1, j1, m1 = scan_one(2 * c + 1)
        plsc.store_scatter(jw_buf, [i0], j0, mask=m0)
        plsc.store_scatter(jw_buf, [i1], j1, mask=m1)
        return 0

    lax.fori_loop(0, (n + 2 * L - 1) // (2 * L), scan_pair, 0)

    def comp_body(i, off):
        v = jw_buf[pl.ds(i * L, L)]
        m = v >= 0
        mi = m.astype(jnp.int32)
        pos = off + plsc.cumsum(mi) - 1
        t = lo + i * L + lane
        plsc.store_scatter(tl128, [pos >> 7, pos & 127], t, mask=m)
        plsc.store_scatter(jl128, [pos >> 7, pos & 127], v, mask=m)
        plsc.store_scatter(tl16, [pos >> 4, pos & 15], t, mask=m)
        plsc.store_scatter(jl16, [pos >> 4, pos & 15], v, mask=m)
        return off + jnp.sum(mi)

    cnt = lax.fori_loop(0, tpb // L, comp_body, jnp.int32(0))

    cntbuf[...] = _splat(cnt)
    pltpu.sync_copy(tl128, tl128_hbm.at[wid])
    pltpu.sync_copy(jl128, jl128_hbm.at[wid])
    pltpu.sync_copy(tl16, tl16_hbm.at[wid])
    pltpu.sync_copy(jl16, jl16_hbm.at[wid])
    pltpu.sync_copy(cntbuf, cnt_hbm.at[wid])


def _invert(idx, n, tp, zbase):
    tpb = tp // NW
    ch128 = (tpb + 127) // 128
    ch16 = tpb // L
    body = functools.partial(_invert_body, n=n, tpb=tpb, ch128=ch128,
                             ch16=ch16, zbase=zbase)
    i32 = jnp.int32
    return pl.kernel(
        body,
        out_type=(
            jax.ShapeDtypeStruct((NW, ch128, 128), i32),
            jax.ShapeDtypeStruct((NW, ch128, 128), i32),
            jax.ShapeDtypeStruct((NW, ch16, L), i32),
            jax.ShapeDtypeStruct((NW, ch16, L), i32),
            jax.ShapeDtypeStruct((NW, L), i32),
        ),
        mesh=plsc.VectorSubcoreMesh(**_MESH),
        scratch_types=[
            pltpu.VMEM((25024,), i32),
            pltpu.VMEM((tpb,), i32),
            pltpu.VMEM((ch128, 128), i32),
            pltpu.VMEM((ch128, 128), i32),
            pltpu.VMEM((ch16, L), i32),
            pltpu.VMEM((ch16, L), i32),
            pltpu.VMEM((L,), i32),
        ],
        compiler_params=pltpu.CompilerParams(needs_layout_passes=False, use_tc_tiling_on_sc=True),
    )(idx)


def _rmw_body(tl128_hbm, jl128_hbm, tl16_hbm, jl16_hbm, cnt_hbm, *gh_args,
              ch128, ch16, npairs):
    gh_pairs = [(gh_args[2 * i], gh_args[2 * i + 1]) for i in range(npairs)]
    (tl128, jl128, tl16, jl16, cntbuf, bufG, bufH, bufG2, bufH2,
     semG, semH, semS, semG2, semH2, semS2) = gh_args[2 * npairs:]
    wid = _wid()
    lane = _lane()
    pltpu.sync_copy(tl128_hbm.at[wid], tl128)
    pltpu.sync_copy(jl128_hbm.at[wid], jl128)
    pltpu.sync_copy(tl16_hbm.at[wid], tl16)
    pltpu.sync_copy(jl16_hbm.at[wid], jl16)
    pltpu.sync_copy(cnt_hbm.at[wid], cntbuf)
    cnt = jnp.max(cntbuf[...])
    n128 = cnt >> 7
    n16 = (cnt & 127) >> 4
    rem = cnt & 15

    def add_rows(bh, bg, nrows):
        def row_body(r, _):
            for c in range(8):
                s = pl.ds(c * L, L)
                bh[r, s] = bh[r, s] + bg[r, s]
            return 0
        lax.fori_loop(0, nrows, row_body, 0)

    for g_hbm, h_ref in gh_pairs:
        _rmw_one(g_hbm, h_ref, tl128, jl128, tl16, jl16, lane, n128, n16, rem,
                 bufG, bufH, bufG2, bufH2, semG, semH, semS, semG2, semH2,
                 semS2, add_rows)


def _rmw_one(g_hbm, h_ref, tl128, jl128, tl16, jl16, lane, n128, n16, rem,
             bufG, bufH, bufG2, bufH2, semG, semH, semS, semG2, semH2, semS2,
             add_rows):
    def big_pair(p, _):
        k0 = 2 * p
        k1 = 2 * p + 1
        gh0 = pltpu.async_copy(g_hbm.at[jl128.at[k0]], bufG, semG)
        hh0 = pltpu.async_copy(h_ref.at[tl128.at[k0]], bufH, semH)
        gh1 = pltpu.async_copy(g_hbm.at[jl128.at[k1]], bufG2, semG2)
        hh1 = pltpu.async_copy(h_ref.at[tl128.at[k1]], bufH2, semH2)
        gh0.wait()
        hh0.wait()
        add_rows(bufH, bufG, 128)
        sc0 = pltpu.async_copy(bufH, h_ref.at[tl128.at[k0]], semS)
        gh1.wait()
        hh1.wait()
        add_rows(bufH2, bufG2, 128)
        sc1 = pltpu.async_copy(bufH2, h_ref.at[tl128.at[k1]], semS2)
        sc0.wait()
        sc1.wait()
        return 0

    lax.fori_loop(0, n128 >> 1, big_pair, 0)

    def big_last(k, _):
        gh = pltpu.async_copy(g_hbm.at[jl128.at[k]], bufG, semG)
        hh = pltpu.async_copy(h_ref.at[tl128.at[k]], bufH, semH)
        gh.wait()
        hh.wait()
        add_rows(bufH, bufG, 128)
        pltpu.async_copy(bufH, h_ref.at[tl128.at[k]], semS).wait()
        return 0

    lax.fori_loop(n128 & ~1, n128, big_last, 0)

    def mid_chunk(i, _):
        k = n128 * 8 + i
        gh = pltpu.async_copy(g_hbm.at[jl16.at[k]], bufG.at[pl.ds(0, L)],
                              semG)
        hh = pltpu.async_copy(h_ref.at[tl16.at[k]], bufH.at[pl.ds(0, L)],
                              semH)
        gh.wait()
        hh.wait()
        add_rows(bufH, bufG, L)
        pltpu.async_copy(bufH.at[pl.ds(0, L)], h_ref.at[tl16.at[k]],
                         semS).wait()
        return 0

    lax.fori_loop(0, n16, mid_chunk, 0)

    ktail = n128 * 8 + n16
    tvec = plsc.load_gather(tl16, [_splat(ktail), lane])
    jvec = plsc.load_gather(jl16, [_splat(ktail), lane])

    def scalar_at(vec, e):
        return jnp.max(jnp.where(lane == e, vec, -1))

    for e in range(15):
        @pl.when(e < rem)
        def _start():
            j_e = scalar_at(jvec, e)
            t_e = scalar_at(tvec, e)
            pltpu.make_async_copy(g_hbm.at[pl.ds(j_e, 1)],
                                  bufG.at[pl.ds(e, 1)], semG).start()
            pltpu.make_async_copy(h_ref.at[pl.ds(t_e, 1)],
                                  bufH.at[pl.ds(e, 1)], semH).start()

    for e in range(15):
        @pl.when(e < rem)
        def _apply():
            pltpu.make_async_copy(g_hbm.at[pl.ds(0, 1)],
                                  bufG.at[pl.ds(e, 1)], semG).wait()
            pltpu.make_async_copy(g_hbm.at[pl.ds(0, 1)],
                                  bufH.at[pl.ds(e, 1)], semH).wait()
            for c in range(8):
                s = pl.ds(c * L, L)
                bufH[e, s] = bufH[e, s] + bufG[e, s]
            t_e = scalar_at(tvec, e)
            pltpu.make_async_copy(bufH.at[pl.ds(e, 1)],
                                  h_ref.at[pl.ds(t_e, 1)], semS).start()

    for e in range(15):
        @pl.when(e < rem)
        def _drain():
            pltpu.make_async_copy(bufH.at[pl.ds(e, 1)],
                                  h_ref.at[pl.ds(0, 1)], semS).wait()


def _rmw(lists, gh_pairs, tp):
    tpb = tp // NW
    ch128 = (tpb + 127) // 128
    ch16 = tpb // L
    body = functools.partial(_rmw_body, ch128=ch128, ch16=ch16,
                             npairs=len(gh_pairs))
    i32 = jnp.int32
    flat = [x for gh in gh_pairs for x in gh]
    pl.kernel(
        body,
        out_type=(),
        mesh=plsc.VectorSubcoreMesh(**_MESH),
        scratch_types=[
            pltpu.VMEM((ch128, 128), i32),
            pltpu.VMEM((ch128, 128), i32),
            pltpu.VMEM((ch16, L), i32),
            pltpu.VMEM((ch16, L), i32),
            pltpu.VMEM((L,), i32),
            pltpu.VMEM((128, D), jnp.float32),
            pltpu.VMEM((128, D), jnp.float32),
            pltpu.VMEM((128, D), jnp.float32),
            pltpu.VMEM((128, D), jnp.float32),
            pltpu.SemaphoreType.DMA,
            pltpu.SemaphoreType.DMA,
            pltpu.SemaphoreType.DMA,
            pltpu.SemaphoreType.DMA,
            pltpu.SemaphoreType.DMA,
            pltpu.SemaphoreType.DMA,
        ],
        compiler_params=pltpu.CompilerParams(needs_layout_passes=False, use_tc_tiling_on_sc=True),
    )(*lists, *flat)


def _prep_body(w0, b0, bup0, w0b, w1, b1, bup1, w1b, w1t, w0t,
               w0e, b0e, w1e, b1e, wc):
    f32 = jnp.float32
    w0e[...] = w0[...] + jnp.dot(w0[...], w0b[...], preferred_element_type=f32)
    b0e[...] = b0[...] + jnp.dot(b0[...], w0b[...], preferred_element_type=f32) + bup0[...]
    w1e[...] = w1[...] + jnp.dot(w1[...], w1b[...], preferred_element_type=f32)
    b1e[...] = b1[...] + jnp.dot(b1[...], w1b[...], preferred_element_type=f32) + bup1[...]
    wc[...] = jnp.dot(w1t[...], w0t[...], preferred_element_type=f32)


def _prep(W0, b0r, bup0r, Wup0b, W1, b1r, bup1r, Wup1b, Wup1t, Wup0t):
    f32 = jnp.float32
    return pl.pallas_call(
        _prep_body,
        out_shape=(
            jax.ShapeDtypeStruct((D, D), f32),
            jax.ShapeDtypeStruct((1, D), f32),
            jax.ShapeDtypeStruct((D, D), f32),
            jax.ShapeDtypeStruct((1, D), f32),
            jax.ShapeDtypeStruct((D, D), f32),
        ),
    )(W0, b0r, bup0r, Wup0b, W1, b1r, bup1r, Wup1b, Wup1t, Wup0t)


def _tc2_body(hn2_ref, w2_ref, b2_ref, wt_ref, wc_ref, h2_ref, g2_ref,
              gc2_ref):
    g = pl.program_id(0)
    rows = g * BLK + lax.broadcasted_iota(jnp.int32, (BLK, 1), 0)
    h = jnp.dot(hn2_ref[...], w2_ref[...],
                preferred_element_type=jnp.float32) + b2_ref[...]
    h2_ref[...] = h
    gv = jnp.dot(h, wt_ref[...], preferred_element_type=jnp.float32)
    g2_ref[...] = jnp.where(rows < N2, gv, 0.0)
    gcv = jnp.dot(h, wc_ref[...], preferred_element_type=jnp.float32)
    gc2_ref[...] = jnp.where(rows < N2, gcv, 0.0)


def _tc2(hn2, W2, b2r, Wup1t, Wc):
    nb = N2P // BLK
    return pl.pallas_call(
        _tc2_body,
        grid=(nb,),
        in_specs=[
            pl.BlockSpec((BLK, D), lambda i: (i, 0)),
            pl.BlockSpec((D, D), lambda i: (0, 0)),
            pl.BlockSpec((1, D), lambda i: (0, 0)),
            pl.BlockSpec((D, D), lambda i: (0, 0)),
            pl.BlockSpec((D, D), lambda i: (0, 0)),
        ],
        out_specs=(
            pl.BlockSpec((BLK, D), lambda i: (i, 0)),
            pl.BlockSpec((BLK, D), lambda i: (i, 0)),
            pl.BlockSpec((BLK, D), lambda i: (i, 0)),
        ),
        out_shape=(
            jax.ShapeDtypeStruct((N2, D), jnp.float32),
            jax.ShapeDtypeStruct((N2P, D), jnp.float32),
            jax.ShapeDtypeStruct((N2P, D), jnp.float32),
        ),
    )(hn2, W2, b2r, Wup1t, Wc)


def _dense_body(hn_ref, w_ref, b_ref, out_ref):
    out_ref[...] = jnp.dot(hn_ref[...], w_ref[...],
                           preferred_element_type=jnp.float32) + b_ref[...]


def _dense(hn, We, be, nrows, blk=BLK):
    return pl.pallas_call(
        _dense_body,
        grid=(nrows // blk,),
        in_specs=[
            pl.BlockSpec((blk, D), lambda i: (i, 0)),
            pl.BlockSpec((D, D), lambda i: (0, 0)),
            pl.BlockSpec((1, D), lambda i: (0, 0)),
        ],
        out_specs=pl.BlockSpec((blk, D), lambda i: (i, 0)),
        out_shape=jax.ShapeDtypeStruct((nrows, D), jnp.float32),
    )(hn, We, be)


def _dense1g_body(hn_ref, w_ref, b_ref, wt_ref, h1_ref, g1_ref):
    nb = N1P // BLK
    r = (nb - 1) - pl.program_id(0)
    rows = r * BLK + lax.broadcasted_iota(jnp.int32, (BLK, 1), 0)
    h = jnp.dot(hn_ref[...], w_ref[...],
                preferred_element_type=jnp.float32) + b_ref[...]
    h1_ref[...] = h
    gv = jnp.dot(h, wt_ref[...], preferred_element_type=jnp.float32)
    g1_ref[...] = jnp.where(rows < N1, gv, 0.0)


def _dense1g(hn1, We, be, Wup0t):
    nb = N1P // BLK
    rev = nb - 1
    return pl.pallas_call(
        _dense1g_body,
        grid=(nb,),
        in_specs=[
            pl.BlockSpec((BLK, D), lambda i: (jnp.minimum(rev - i, rev - 1), 0)),
            pl.BlockSpec((D, D), lambda i: (0, 0)),
            pl.BlockSpec((1, D), lambda i: (0, 0)),
            pl.BlockSpec((D, D), lambda i: (0, 0)),
        ],
        out_specs=(
            pl.BlockSpec((BLK, D), lambda i: (jnp.minimum(rev - i, rev - 1), 0)),
            pl.BlockSpec((BLK, D), lambda i: (rev - i, 0)),
        ),
        out_shape=(
            jax.ShapeDtypeStruct((N1, D), jnp.float32),
            jax.ShapeDtypeStruct((N1P, D), jnp.float32),
        ),
    )(hn1, We, be, Wup0t)


def kernel(hn0, hn1, hn2, idx1, idx2, W0, b0, W1, b1, W2, b2, Wup0, bup0,
           Wup1, bup1):
    b0r, b1r, b2r = b0.reshape(1, D), b1.reshape(1, D), b2.reshape(1, D)
    bup0r, bup1r = bup0.reshape(1, D), bup1.reshape(1, D)
    Wup0t, Wup0b = Wup0[:D], Wup0[D:]
    Wup1t, Wup1b = Wup1[:D], Wup1[D:]

    W0e, b0e, W1e, b1e, Wc = _prep(W0, b0r, bup0r, Wup0b, W1, b1r, bup1r,
                                   Wup1b, Wup1t, Wup0t)

    lists2 = _invert(idx2, N2, T1P, zbase=N2)
    lists1 = _invert(idx1, N1, T0P, zbase=N1)

    h2, g2pad, gc2pad = _tc2(hn2, W2, b2r, Wup1t, Wc)
    h1d, g1d = _dense1g(hn1, W1e, b1e, Wup0t)
    h0d = _dense(hn0, W0e, b0e, N0, blk=4000)

    h1_ref = jax.new_ref(h1d)
    g1_ref = jax.new_ref(g1d)
    _rmw(lists2, [(g2pad, h1_ref), (gc2pad, g1_ref)], T1P)
    h1 = jax.freeze(h1_ref)
    g1pad = jax.freeze(g1_ref)

    h0_ref = jax.new_ref(h0d)
    _rmw(lists1, [(g1pad, h0_ref)], T0P)
    h0 = jax.freeze(h0_ref)

    return (h0, h1, h2)
